# speedup vs baseline: 2.1445x; 2.1445x over previous
import functools

import jax
import jax.numpy as jnp
from jax.experimental import pallas as pl
from jax.experimental.pallas import tpu as pltpu

_T = 64
_MB = 128


def _count_norm_kernel(ql_ref, tl_ref, out_ref):
    T = ql_ref.shape[0]
    Mb = ql_ref.shape[1]
    Npad = tl_ref.shape[1]

    def body(t, acc):
        qlt = ql_ref[t, :]
        tlt = tl_ref[t, :]
        eq = qlt[:, None] == tlt[None, :]
        return acc + eq.astype(jnp.float32)

    acc = jax.lax.fori_loop(0, T, body, jnp.zeros((Mb, Npad), jnp.float32))
    s = jnp.sum(acc, axis=1, keepdims=True)
    out_ref[...] = acc / (s + T * 1e-8)


def kernel(X, query_leaves, train_leaves):
    del X
    T, M = query_leaves.shape
    N = train_leaves.shape[1]
    Npad = (N + 127) // 128 * 128
    tl = jnp.pad(train_leaves, ((0, 0), (0, Npad - N)), constant_values=-1)

    grid = (M // _MB,)
    out = pl.pallas_call(
        _count_norm_kernel,
        grid=grid,
        in_specs=[
            pl.BlockSpec((T, _MB), lambda i: (0, i)),
            pl.BlockSpec((T, Npad), lambda i: (0, 0)),
        ],
        out_specs=pl.BlockSpec((_MB, Npad), lambda i: (i, 0)),
        out_shape=jax.ShapeDtypeStruct((M, Npad), jnp.float32),
        compiler_params=pltpu.CompilerParams(
            dimension_semantics=("arbitrary",),
        ),
    )(query_leaves, tl)
    return out[:, :N]


# speedup vs baseline: 6.0341x; 2.8138x over previous
import functools

import jax
import jax.numpy as jnp
from jax import lax
from jax.experimental import pallas as pl
from jax.experimental.pallas import tpu as pltpu
from jax.experimental.pallas import tpu_sc as plsc

_T = 64
_M = 1024
_N = 20000
_L = 512

_NTILES = 32
_TREES_PER_TILE = _T // _NTILES
_W = 16
_NSTRIPS = _N // _W


def _iota16():
    return lax.iota(jnp.int32, 16)


def _wid():
    return lax.axis_index("c") * 16 + lax.axis_index("s")


def _csr_body(ql_hbm, tl_hbm, qperm_hbm, spk_hbm, thist_hbm,
              qlv, tlv, cnt, seg, out1k, out512):
    w = _wid()
    lanes = _iota16()

    def do_tree(i, _):
        t = w * _TREES_PER_TILE + i
        pltpu.sync_copy(ql_hbm.at[t], qlv)
        pltpu.sync_copy(tl_hbm.at[t], tlv)

        def zero_blk(b, _):
            cnt[pl.ds(b * 16, 16)] = jnp.zeros((16,), jnp.int32)
            return 0
        lax.fori_loop(0, _L, zero_blk, 0)

        def qhist(k, _):
            lv = plsc.load_gather(qlv, [lanes * (_M // 16) + k])
            a = lv * 16 + lanes
            c = plsc.load_gather(cnt, [a])
            plsc.store_scatter(cnt, [a], c + 1)
            return 0
        lax.fori_loop(0, _M // 16, qhist, 0)

        def scan_blk(b, carry):
            v = cnt[pl.ds(b * 16, 16)]
            inc = plsc.cumsum(v)
            cnt[pl.ds(b * 16, 16)] = inc - v + carry
            return carry + jnp.max(inc)
        total = lax.fori_loop(0, _L, scan_blk, jnp.int32(0))
        seg[pl.ds(_L * 16, 16)] = jnp.full((16,), total, jnp.int32)

        def copy_sentinelless(b, _):
            seg[pl.ds(b * 16, 16)] = cnt[pl.ds(b * 16, 16)]
            return 0
        lax.fori_loop(0, _L, copy_sentinelless, 0)

        def spk_blk(b, _):
            l16 = b * 16 + lanes
            starts = plsc.load_gather(seg, [l16 * 16])
            ends = plsc.load_gather(seg, [l16 * 16 + 16])
            out512[pl.ds(b * 16, 16)] = (
                lax.shift_left(starts, 12) | (ends - starts))
            return 0
        lax.fori_loop(0, _L // 16, spk_blk, 0)
        pltpu.sync_copy(out512, spk_hbm.at[t])

        def qscat(k, _):
            mv = lanes * (_M // 16) + k
            lv = plsc.load_gather(qlv, [mv])
            a = lv * 16 + lanes
            cur = plsc.load_gather(seg, [a])
            plsc.store_scatter(out1k, [cur], mv)
            plsc.store_scatter(seg, [a], cur + 1)
            return 0
        lax.fori_loop(0, _M // 16, qscat, 0)
        pltpu.sync_copy(out1k, qperm_hbm.at[t])

        lax.fori_loop(0, _L, zero_blk, 0)

        def thist_step(k, _):
            lv = plsc.load_gather(tlv, [lanes * (_N // 16) + k])
            a = lv * 16 + lanes
            c = plsc.load_gather(cnt, [a])
            plsc.store_scatter(cnt, [a], c + 1)
            return 0
        lax.fori_loop(0, _N // 16, thist_step, 0)

        def tfold(b, _):
            l16 = b * 16 + lanes
            acc = plsc.load_gather(cnt, [l16 * 16])
            def add_lane(j, acc):
                return acc + plsc.load_gather(cnt, [l16 * 16 + j])
            acc = lax.fori_loop(1, 16, add_lane, acc)
            out512[pl.ds(b * 16, 16)] = acc
            return 0
        lax.fori_loop(0, _L // 16, tfold, 0)
        pltpu.sync_copy(out512, thist_hbm.at[t])
        return 0

    lax.fori_loop(0, _TREES_PER_TILE, do_tree, 0)


def _rowscale_body(ql_hbm, thist_hbm, rs_hbm, qlb, th, out32):
    w = _wid()
    mchunk = _M // _NTILES
    pltpu.sync_copy(thist_hbm, th)
    pltpu.sync_copy(ql_hbm.at[:, pl.ds(w * mchunk, mchunk)], qlb)

    def per_half(j):
        def per_tree(t, acc):
            lv = qlb[t, pl.ds(j * 16, 16)]
            h = plsc.load_gather(th, [jnp.full((16,), t, jnp.int32), lv])
            return acc + h
        acc = lax.fori_loop(0, _T, per_tree, jnp.zeros((16,), jnp.int32))
        denom = acc.astype(jnp.float32) + jnp.float32(_T * 1e-8)
        out32[pl.ds(j * 16, 16)] = jnp.float32(1.0) / denom

    per_half(0)
    per_half(1)
    pltpu.sync_copy(out32, rs_hbm.at[pl.ds(w * mchunk, mchunk)])


def _join_body(tl_hbm, qperm_hbm, spk_hbm, rs_hbm, out_hbm,
               qperm, spk, rs, acc, tlb):
    w = _wid()
    lanes = _iota16()
    pltpu.sync_copy(qperm_hbm, qperm)
    pltpu.sync_copy(spk_hbm, spk)
    pltpu.sync_copy(rs_hbm, rs)

    nstrips = _NSTRIPS // _NTILES + jnp.where(
        w < _NSTRIPS % _NTILES, 1, 0)

    def do_strip(si, _):
        strip = w + si * _NTILES
        c0 = strip * _W
        pltpu.sync_copy(tl_hbm.at[:, pl.ds(c0, _W)], tlb)

        def zero_rows(r, _):
            for rr in range(8):
                acc[r * 8 + rr, :] = jnp.zeros((16,), jnp.float32)
            return 0
        lax.fori_loop(0, _M // 8, zero_rows, 0)

        def per_tree(t, _):
            tlv = tlb[t, :]
            sp = plsc.load_gather(spk, [jnp.full((16,), t, jnp.int32), tlv])
            start = lax.shift_right_logical(sp, 12)
            ln = sp & 0xFFF
            mx = jnp.max(ln)
            trow = jnp.full((16,), t, jnp.int32)

            def expand(k, _):
                msk = ln > k
                idx = jnp.where(msk, start + k, 0)
                mm = plsc.load_gather(qperm, [trow, idx])
                val = plsc.load_gather(rs, [mm])
                plsc.addupdate_scatter(acc, [mm, lanes], val, mask=msk)
                return 0
            lax.fori_loop(0, mx, expand, 0)
            return 0
        lax.fori_loop(0, _T, per_tree, 0)
        pltpu.sync_copy(acc, out_hbm.at[:, pl.ds(c0, _W)])
        return 0

    lax.fori_loop(0, nstrips, do_strip, 0)


def _sc_pipeline(ql, tl):
    mesh = plsc.VectorSubcoreMesh(core_axis_name="c", subcore_axis_name="s")

    csr = pl.kernel(
        _csr_body,
        mesh=mesh,
        compiler_params=pltpu.CompilerParams(needs_layout_passes=False, use_tc_tiling_on_sc=False),
        out_type=(
            jax.ShapeDtypeStruct((_T, _M), jnp.int32),
            jax.ShapeDtypeStruct((_T, _L), jnp.int32),
            jax.ShapeDtypeStruct((_T, _L), jnp.int32),
        ),
        scratch_types=[
            pltpu.VMEM((_M,), jnp.int32),
            pltpu.VMEM((_N,), jnp.int32),
            pltpu.VMEM((_L * 16,), jnp.int32),
            pltpu.VMEM((_L * 16 + 16,), jnp.int32),
            pltpu.VMEM((_M,), jnp.int32),
            pltpu.VMEM((_L,), jnp.int32),
        ],
    )
    qperm, spk, thist = csr(ql, tl)

    rowscale = pl.kernel(
        _rowscale_body,
        mesh=mesh,
        compiler_params=pltpu.CompilerParams(needs_layout_passes=False, use_tc_tiling_on_sc=False),
        out_type=jax.ShapeDtypeStruct((_M,), jnp.float32),
        scratch_types=[
            pltpu.VMEM((_T, _M // _NTILES), jnp.int32),
            pltpu.VMEM((_T, _L), jnp.int32),
            pltpu.VMEM((_M // _NTILES,), jnp.float32),
        ],
    )(ql, thist)

    out = pl.kernel(
        _join_body,
        mesh=mesh,
        compiler_params=pltpu.CompilerParams(needs_layout_passes=False, use_tc_tiling_on_sc=False),
        out_type=jax.ShapeDtypeStruct((_M, _N), jnp.float32),
        scratch_types=[
            pltpu.VMEM((_T, _M), jnp.int32),
            pltpu.VMEM((_T, _L), jnp.int32),
            pltpu.VMEM((_M,), jnp.float32),
            pltpu.VMEM((_M, _W), jnp.float32),
            pltpu.VMEM((_T, _W), jnp.int32),
        ],
    )(tl, qperm, spk, rowscale)
    return out


def kernel(X, query_leaves, train_leaves):
    del X
    return _sc_pipeline(query_leaves, train_leaves)


# speedup vs baseline: 7.2885x; 1.2079x over previous
import functools

import jax
import jax.numpy as jnp
from jax import lax
from jax.experimental import pallas as pl
from jax.experimental.pallas import tpu as pltpu
from jax.experimental.pallas import tpu_sc as plsc

_T = 64
_M = 1024
_N = 20000
_L = 512

_NTILES = 32
_TREES_PER_TILE = _T // _NTILES
_W = 16
_NSTRIPS = _N // _W
_SPW = 272


def _iota16():
    return lax.iota(jnp.int32, 16)


def _wid():
    return lax.axis_index("c") * 16 + lax.axis_index("s")


def _csr_body(ql_hbm, tl_hbm, qperm_hbm, spk_hbm, thist_hbm,
              qlv, tlv, cnt, cnt2, seg, out1k, out512, sbuf):
    w = _wid()
    lanes = _iota16()

    def do_tree(i, _):
        t = w * _TREES_PER_TILE + i
        pltpu.sync_copy(ql_hbm.at[t], qlv)
        pltpu.sync_copy(tl_hbm.at[t], tlv)

        def zero_blk(b, _):
            cnt[pl.ds(b * 16, 16)] = jnp.zeros((16,), jnp.int32)
            return 0
        lax.fori_loop(0, _L, zero_blk, 0)

        def qhist(k, _):
            lv = plsc.load_gather(qlv, [lanes * (_M // 16) + k])
            a = lv * 16 + lanes
            c = plsc.load_gather(cnt, [a])
            plsc.store_scatter(cnt, [a], c + 1)
            return 0
        lax.fori_loop(0, _M // 16, qhist, 0)

        def scan_blk(b, carry):
            v = cnt[pl.ds(b * 16, 16)]
            inc = plsc.cumsum(v)
            cnt[pl.ds(b * 16, 16)] = inc - v + carry
            return carry + jnp.max(inc)
        total = lax.fori_loop(0, _L, scan_blk, jnp.int32(0))
        seg[pl.ds(_L * 16, 16)] = jnp.full((16,), total, jnp.int32)

        def copy_sentinelless(b, _):
            seg[pl.ds(b * 16, 16)] = cnt[pl.ds(b * 16, 16)]
            return 0
        lax.fori_loop(0, _L, copy_sentinelless, 0)

        def spk_blk(b, _):
            wj = b * 16 + lanes
            e0 = jnp.minimum(wj * 2, _L) * 16
            e1 = jnp.minimum(wj * 2 + 1, _L) * 16
            lo = plsc.load_gather(seg, [e0])
            hi = plsc.load_gather(seg, [e1])
            sbuf[pl.ds(b * 16, 16)] = lo | lax.shift_left(hi, 16)
            return 0
        lax.fori_loop(0, _SPW // 16, spk_blk, 0)
        pltpu.sync_copy(sbuf, spk_hbm.at[t])

        def qscat(k, _):
            mv = lanes * (_M // 16) + k
            lv = plsc.load_gather(qlv, [mv])
            a = lv * 16 + lanes
            cur = plsc.load_gather(seg, [a])
            plsc.store_scatter(out1k, [cur], mv)
            plsc.store_scatter(seg, [a], cur + 1)
            return 0
        lax.fori_loop(0, _M // 16, qscat, 0)
        pltpu.sync_copy(out1k, qperm_hbm.at[t])

        def zero_blk2(b, _):
            cnt2[pl.ds(b * 16, 16)] = jnp.zeros((16,), jnp.int32)
            return 0
        lax.fori_loop(0, _L * 2, zero_blk2, 0)

        def thist_step(k2, _):
            k = k2 * 2
            lv0 = plsc.load_gather(tlv, [lanes * (_N // 16) + k])
            lv1 = plsc.load_gather(tlv, [lanes * (_N // 16) + k + 1])
            a0 = lv0 * 32 + lanes * 2
            a1 = lv1 * 32 + lanes * 2 + 1
            c0 = plsc.load_gather(cnt2, [a0])
            c1 = plsc.load_gather(cnt2, [a1])
            plsc.store_scatter(cnt2, [a0], c0 + 1)
            plsc.store_scatter(cnt2, [a1], c1 + 1)
            return 0
        lax.fori_loop(0, _N // 32, thist_step, 0)

        def tfold(b, _):
            l16 = b * 16 + lanes
            acc = plsc.load_gather(cnt2, [l16 * 32])
            def add_lane(j, acc):
                return acc + plsc.load_gather(cnt2, [l16 * 32 + j])
            acc = lax.fori_loop(1, 32, add_lane, acc)
            out512[pl.ds(b * 16, 16)] = acc
            return 0
        lax.fori_loop(0, _L // 16, tfold, 0)
        pltpu.sync_copy(out512, thist_hbm.at[t])
        return 0

    lax.fori_loop(0, _TREES_PER_TILE, do_tree, 0)


def _rowscale_body(ql_hbm, thist_hbm, rs_hbm, qlb, th, out32):
    w = _wid()
    mchunk = _M // _NTILES
    pltpu.sync_copy(thist_hbm, th)
    pltpu.sync_copy(ql_hbm.at[:, pl.ds(w * mchunk, mchunk)], qlb)

    def per_half(j):
        def per_tree(t, acc):
            lv = qlb[t, pl.ds(j * 16, 16)]
            h = plsc.load_gather(th, [jnp.full((16,), t, jnp.int32), lv])
            return acc + h
        acc = lax.fori_loop(0, _T, per_tree, jnp.zeros((16,), jnp.int32))
        denom = acc.astype(jnp.float32) + jnp.float32(_T * 1e-8)
        out32[pl.ds(j * 16, 16)] = jnp.float32(1.0) / denom

    per_half(0)
    per_half(1)
    pltpu.sync_copy(out32, rs_hbm.at[pl.ds(w * mchunk, mchunk)])


def _qdata_body(qperm_hbm, rs_hbm, qdata_hbm, qpv, rsv, qdv):
    w = _wid()
    pltpu.sync_copy(rs_hbm, rsv)

    def do_tree(i, _):
        t = w * _TREES_PER_TILE + i
        pltpu.sync_copy(qperm_hbm.at[t], qpv)

        def step(j, _):
            mv = qpv[pl.ds(j * 16, 16)]
            rb = plsc.bitcast(plsc.load_gather(rsv, [mv]), jnp.int32)
            rb = (rb + 0x8000) & jnp.int32(-65536)
            qdv[pl.ds(j * 16, 16)] = rb | mv
            return 0
        lax.fori_loop(0, _M // 16, step, 0)
        pltpu.sync_copy(qdv, qdata_hbm.at[t])
        return 0

    lax.fori_loop(0, _TREES_PER_TILE, do_tree, 0)


def _join_body(tl_hbm, qdata_hbm, spk_hbm, out_hbm,
               qdata, spk, acc, tlb, osem, tsem):
    w = _wid()
    lanes = _iota16()
    pltpu.sync_copy(qdata_hbm, qdata)
    pltpu.sync_copy(spk_hbm, spk)

    npass = (_NSTRIPS + _NTILES - 1) // _NTILES

    def strip_of(si):
        return w + si * _NTILES

    def tl_copy(si):
        return pltpu.make_async_copy(
            tl_hbm.at[:, pl.ds(strip_of(si) * _W, _W)], tlb, tsem)

    def out_copy(si, b):
        return pltpu.make_async_copy(
            acc.at[b], out_hbm.at[:, pl.ds(strip_of(si) * _W, _W)],
            osem.at[b])


    def do_pass(p, _):
        for h in range(2):
            si = p * 2 + h
            b = h

            @pl.when(strip_of(si) < _NSTRIPS)
            def _():
                acc_b = acc.at[b]

                tl_copy(si).start()

                @pl.when(si >= 2)
                def _():
                    out_copy(si - 2, b).wait()

                def zero_rows(r, _):
                    for rr in range(8):
                        acc_b[r * 8 + rr, :] = jnp.zeros((16,), jnp.float32)
                    return 0
                lax.fori_loop(0, _M // 8, zero_rows, 0)
                tl_copy(si).wait()

                def per_tree(t, _):
                    tlv = tlb[t, :]
                    trow = jnp.full((16,), t, jnp.int32)
                    wv = lax.shift_right_logical(tlv, 1)
                    odd = (tlv & 1) > 0
                    u0 = plsc.load_gather(spk, [trow, wv])
                    u1 = plsc.load_gather(spk, [trow, wv + 1])
                    lo0 = u0 & 0xFFFF
                    hi0 = lax.shift_right_logical(u0, 16)
                    lo1 = u1 & 0xFFFF
                    start = jnp.where(odd, hi0, lo0)
                    ln = jnp.where(odd, lo1, hi0) - start
                    mx = jnp.max(ln)

                    def expand(k2, _):
                        k = k2 * 2
                        msk0 = ln > k
                        msk1 = ln > k + 1
                        idx0 = jnp.where(msk0, start + k, 0)
                        idx1 = jnp.where(msk1, start + k + 1, 0)
                        q0 = plsc.load_gather(qdata, [trow, idx0])
                        q1 = plsc.load_gather(qdata, [trow, idx1])
                        mm0 = q0 & 0xFFFF
                        mm1 = q1 & 0xFFFF
                        v0 = plsc.bitcast(q0 & jnp.int32(-65536), jnp.float32)
                        v1 = plsc.bitcast(q1 & jnp.int32(-65536), jnp.float32)
                        plsc.addupdate_scatter(
                            acc_b, [mm0, lanes], v0, mask=msk0)
                        plsc.addupdate_scatter(
                            acc_b, [mm1, lanes], v1, mask=msk1)
                        return 0
                    lax.fori_loop(0, (mx + 1) // 2, expand, 0)
                    return 0
                lax.fori_loop(0, _T, per_tree, 0)
                out_copy(si, b).start()
        return 0

    lax.fori_loop(0, (npass + 1) // 2, do_pass, 0)

    nstrips = _NSTRIPS // _NTILES + jnp.where(w < _NSTRIPS % _NTILES, 1, 0)

    @pl.when(nstrips >= 2)
    def _():
        out_copy(nstrips - 2, (nstrips - 2) % 2).wait()

    @pl.when(nstrips >= 1)
    def _():
        out_copy(nstrips - 1, (nstrips - 1) % 2).wait()


def _sc_pipeline(ql, tl):
    mesh = plsc.VectorSubcoreMesh(core_axis_name="c", subcore_axis_name="s")

    csr = pl.kernel(
        _csr_body,
        mesh=mesh,
        compiler_params=pltpu.CompilerParams(needs_layout_passes=False, use_tc_tiling_on_sc=False),
        out_type=(
            jax.ShapeDtypeStruct((_T, _M), jnp.int32),
            jax.ShapeDtypeStruct((_T, _SPW), jnp.int32),
            jax.ShapeDtypeStruct((_T, _L), jnp.int32),
        ),
        scratch_types=[
            pltpu.VMEM((_M,), jnp.int32),
            pltpu.VMEM((_N,), jnp.int32),
            pltpu.VMEM((_L * 16,), jnp.int32),
            pltpu.VMEM((_L * 32,), jnp.int32),
            pltpu.VMEM((_L * 16 + 16,), jnp.int32),
            pltpu.VMEM((_M,), jnp.int32),
            pltpu.VMEM((_L,), jnp.int32),
            pltpu.VMEM((_SPW,), jnp.int32),
        ],
    )
    qperm, spk, thist = csr(ql, tl)

    rowscale = pl.kernel(
        _rowscale_body,
        mesh=mesh,
        compiler_params=pltpu.CompilerParams(needs_layout_passes=False, use_tc_tiling_on_sc=False),
        out_type=jax.ShapeDtypeStruct((_M,), jnp.float32),
        scratch_types=[
            pltpu.VMEM((_T, _M // _NTILES), jnp.int32),
            pltpu.VMEM((_T, _L), jnp.int32),
            pltpu.VMEM((_M // _NTILES,), jnp.float32),
        ],
    )(ql, thist)

    qdata = pl.kernel(
        _qdata_body,
        mesh=mesh,
        compiler_params=pltpu.CompilerParams(needs_layout_passes=False, use_tc_tiling_on_sc=False),
        out_type=jax.ShapeDtypeStruct((_T, _M), jnp.int32),
        scratch_types=[
            pltpu.VMEM((_M,), jnp.int32),
            pltpu.VMEM((_M,), jnp.float32),
            pltpu.VMEM((_M,), jnp.int32),
        ],
    )(qperm, rowscale)

    out = pl.kernel(
        _join_body,
        mesh=mesh,
        compiler_params=pltpu.CompilerParams(needs_layout_passes=False, use_tc_tiling_on_sc=False),
        out_type=jax.ShapeDtypeStruct((_M, _N), jnp.float32),
        scratch_types=[
            pltpu.VMEM((_T, _M), jnp.int32),
            pltpu.VMEM((_T, _SPW), jnp.int32),
            pltpu.VMEM((2, _M, _W), jnp.float32),
            pltpu.VMEM((_T, _W), jnp.int32),
            pltpu.SemaphoreType.DMA((2,)),
            pltpu.SemaphoreType.DMA,
        ],
    )(tl, qdata, spk)
    return out


def kernel(X, query_leaves, train_leaves):
    del X
    return _sc_pipeline(query_leaves, train_leaves)


# speedup vs baseline: 7.5847x; 1.0406x over previous
import functools

import jax
import jax.numpy as jnp
from jax import lax
from jax.experimental import pallas as pl
from jax.experimental.pallas import tpu as pltpu
from jax.experimental.pallas import tpu_sc as plsc

_T = 64
_M = 1024
_N = 20000
_L = 512

_NTILES = 32
_TREES_PER_TILE = _T // _NTILES
_W = 16
_NSTRIPS = _N // _W
_SPW = 272


def _iota16():
    return lax.iota(jnp.int32, 16)


def _wid():
    return lax.axis_index("c") * 16 + lax.axis_index("s")


def _csr_body(ql_hbm, tl_hbm, qperm_hbm, spk_hbm, thist_hbm,
              qlv, tlv, cnt, cnt2, seg, out1k, out512, sbuf):
    w = _wid()
    lanes = _iota16()

    def do_tree(i, _):
        t = w * _TREES_PER_TILE + i
        pltpu.sync_copy(ql_hbm.at[t], qlv)
        pltpu.sync_copy(tl_hbm.at[t], tlv)

        def zero_blk(b, _):
            cnt[pl.ds(b * 16, 16)] = jnp.zeros((16,), jnp.int32)
            return 0
        lax.fori_loop(0, _L, zero_blk, 0)

        def qhist(k, _):
            lv = plsc.load_gather(qlv, [lanes * (_M // 16) + k])
            a = lv * 16 + lanes
            c = plsc.load_gather(cnt, [a])
            plsc.store_scatter(cnt, [a], c + 1)
            return 0
        lax.fori_loop(0, _M // 16, qhist, 0)

        def scan_blk(b, carry):
            v = cnt[pl.ds(b * 16, 16)]
            inc = plsc.cumsum(v)
            cnt[pl.ds(b * 16, 16)] = inc - v + carry
            return carry + jnp.max(inc)
        total = lax.fori_loop(0, _L, scan_blk, jnp.int32(0))
        seg[pl.ds(_L * 16, 16)] = jnp.full((16,), total, jnp.int32)

        def copy_sentinelless(b, _):
            seg[pl.ds(b * 16, 16)] = cnt[pl.ds(b * 16, 16)]
            return 0
        lax.fori_loop(0, _L, copy_sentinelless, 0)

        def spk_blk(b, _):
            wj = b * 16 + lanes
            e0 = jnp.minimum(wj * 2, _L) * 16
            e1 = jnp.minimum(wj * 2 + 1, _L) * 16
            lo = plsc.load_gather(seg, [e0])
            hi = plsc.load_gather(seg, [e1])
            sbuf[pl.ds(b * 16, 16)] = lo | lax.shift_left(hi, 16)
            return 0
        lax.fori_loop(0, _SPW // 16, spk_blk, 0)
        pltpu.sync_copy(sbuf, spk_hbm.at[t])

        def qscat(k, _):
            mv = lanes * (_M // 16) + k
            lv = plsc.load_gather(qlv, [mv])
            a = lv * 16 + lanes
            cur = plsc.load_gather(seg, [a])
            plsc.store_scatter(out1k, [cur], mv)
            plsc.store_scatter(seg, [a], cur + 1)
            return 0
        lax.fori_loop(0, _M // 16, qscat, 0)
        pltpu.sync_copy(out1k, qperm_hbm.at[t])

        def zero_blk2(b, _):
            cnt2[pl.ds(b * 16, 16)] = jnp.zeros((16,), jnp.int32)
            return 0
        lax.fori_loop(0, _L * 2, zero_blk2, 0)

        def thist_step(k2, _):
            k = k2 * 2
            lv0 = plsc.load_gather(tlv, [lanes * (_N // 16) + k])
            lv1 = plsc.load_gather(tlv, [lanes * (_N // 16) + k + 1])
            a0 = lv0 * 32 + lanes * 2
            a1 = lv1 * 32 + lanes * 2 + 1
            c0 = plsc.load_gather(cnt2, [a0])
            c1 = plsc.load_gather(cnt2, [a1])
            plsc.store_scatter(cnt2, [a0], c0 + 1)
            plsc.store_scatter(cnt2, [a1], c1 + 1)
            return 0
        lax.fori_loop(0, _N // 32, thist_step, 0)

        def tfold(b, _):
            l16 = b * 16 + lanes
            acc = plsc.load_gather(cnt2, [l16 * 32])
            def add_lane(j, acc):
                return acc + plsc.load_gather(cnt2, [l16 * 32 + j])
            acc = lax.fori_loop(1, 32, add_lane, acc)
            out512[pl.ds(b * 16, 16)] = acc
            return 0
        lax.fori_loop(0, _L // 16, tfold, 0)
        pltpu.sync_copy(out512, thist_hbm.at[t])
        return 0

    lax.fori_loop(0, _TREES_PER_TILE, do_tree, 0)


def _rowscale_body(ql_hbm, thist_hbm, rs_hbm, qlb, th, out32):
    w = _wid()
    mchunk = _M // _NTILES
    pltpu.sync_copy(thist_hbm, th)
    pltpu.sync_copy(ql_hbm.at[:, pl.ds(w * mchunk, mchunk)], qlb)

    def per_half(j):
        def per_tree(t, acc):
            lv = qlb[t, pl.ds(j * 16, 16)]
            h = plsc.load_gather(th, [jnp.full((16,), t, jnp.int32), lv])
            return acc + h
        acc = lax.fori_loop(0, _T, per_tree, jnp.zeros((16,), jnp.int32))
        denom = acc.astype(jnp.float32) + jnp.float32(_T * 1e-8)
        out32[pl.ds(j * 16, 16)] = jnp.float32(1.0) / denom

    per_half(0)
    per_half(1)
    pltpu.sync_copy(out32, rs_hbm.at[pl.ds(w * mchunk, mchunk)])


def _qdata_body(qperm_hbm, rs_hbm, qdata_hbm, qpv, rsv, qdv):
    w = _wid()
    pltpu.sync_copy(rs_hbm, rsv)

    def do_tree(i, _):
        t = w * _TREES_PER_TILE + i
        pltpu.sync_copy(qperm_hbm.at[t], qpv)

        def step(j, _):
            mv = qpv[pl.ds(j * 16, 16)]
            rb = plsc.bitcast(plsc.load_gather(rsv, [mv]), jnp.int32)
            rb = (rb + 0x8000) & jnp.int32(-65536)
            qdv[pl.ds(j * 16, 16)] = rb | mv
            return 0
        lax.fori_loop(0, _M // 16, step, 0)
        pltpu.sync_copy(qdv, qdata_hbm.at[t])
        return 0

    lax.fori_loop(0, _TREES_PER_TILE, do_tree, 0)


def _join_body(tl_hbm, qdata_hbm, spk_hbm, out_hbm,
               qdata, spk, acc, tlb, osem, tsem):
    w = _wid()
    lanes = _iota16()
    pltpu.sync_copy(qdata_hbm, qdata)
    pltpu.sync_copy(spk_hbm, spk)

    npass = (_NSTRIPS + _NTILES - 1) // _NTILES

    def strip_of(si):
        return w + si * _NTILES

    def tl_copy(si):
        return pltpu.make_async_copy(
            tl_hbm.at[:, pl.ds(strip_of(si) * _W, _W)], tlb, tsem)

    def out_copy(si, b):
        return pltpu.make_async_copy(
            acc.at[b], out_hbm.at[:, pl.ds(strip_of(si) * _W, _W)],
            osem.at[b])


    def do_pass(p, _):
        for h in range(2):
            si = p * 2 + h
            b = h

            @pl.when(strip_of(si) < _NSTRIPS)
            def _():
                acc_b = acc.at[b]

                tl_copy(si).start()

                @pl.when(si >= 2)
                def _():
                    out_copy(si - 2, b).wait()

                def zero_rows(r, _):
                    for rr in range(16):
                        acc_b[r * 16 + rr, :] = jnp.zeros((16,), jnp.float32)
                    return 0
                lax.fori_loop(0, _M // 16, zero_rows, 0)
                tl_copy(si).wait()

                def header(t):
                    tlv = tlb[t, :]
                    trow = jnp.full((16,), t, jnp.int32)
                    wv = lax.shift_right_logical(tlv, 1)
                    odd = (tlv & 1) > 0
                    u0 = plsc.load_gather(spk, [trow, wv])
                    u1 = plsc.load_gather(spk, [trow, wv + 1])
                    lo0 = u0 & 0xFFFF
                    hi0 = lax.shift_right_logical(u0, 16)
                    lo1 = u1 & 0xFFFF
                    start = jnp.where(odd, hi0, lo0)
                    ln = jnp.where(odd, lo1, hi0) - start
                    return trow, start, ln

                def expand_tree(trow, start, ln):
                    mx = jnp.max(ln)

                    def expand(k2, _):
                        k = k2 * 2
                        msk0 = ln > k
                        msk1 = ln > k + 1
                        idx0 = jnp.where(msk0, start + k, 0)
                        idx1 = jnp.where(msk1, start + k + 1, 0)
                        q0 = plsc.load_gather(qdata, [trow, idx0])
                        q1 = plsc.load_gather(qdata, [trow, idx1])
                        mm0 = q0 & 0xFFFF
                        mm1 = q1 & 0xFFFF
                        v0 = plsc.bitcast(q0 & jnp.int32(-65536), jnp.float32)
                        v1 = plsc.bitcast(q1 & jnp.int32(-65536), jnp.float32)
                        plsc.addupdate_scatter(
                            acc_b, [mm0, lanes], v0, mask=msk0)
                        plsc.addupdate_scatter(
                            acc_b, [mm1, lanes], v1, mask=msk1)
                        return 0
                    lax.fori_loop(0, (mx + 1) // 2, expand, 0)

                def per_pair(tp, _):
                    h0 = header(tp * 2)
                    h1 = header(tp * 2 + 1)
                    expand_tree(*h0)
                    expand_tree(*h1)
                    return 0
                lax.fori_loop(0, _T // 2, per_pair, 0)
                out_copy(si, b).start()
        return 0

    lax.fori_loop(0, (npass + 1) // 2, do_pass, 0)

    nstrips = _NSTRIPS // _NTILES + jnp.where(w < _NSTRIPS % _NTILES, 1, 0)

    @pl.when(nstrips >= 2)
    def _():
        out_copy(nstrips - 2, (nstrips - 2) % 2).wait()

    @pl.when(nstrips >= 1)
    def _():
        out_copy(nstrips - 1, (nstrips - 1) % 2).wait()


def _sc_pipeline(ql, tl):
    mesh = plsc.VectorSubcoreMesh(core_axis_name="c", subcore_axis_name="s")

    csr = pl.kernel(
        _csr_body,
        mesh=mesh,
        compiler_params=pltpu.CompilerParams(needs_layout_passes=False, use_tc_tiling_on_sc=False),
        out_type=(
            jax.ShapeDtypeStruct((_T, _M), jnp.int32),
            jax.ShapeDtypeStruct((_T, _SPW), jnp.int32),
            jax.ShapeDtypeStruct((_T, _L), jnp.int32),
        ),
        scratch_types=[
            pltpu.VMEM((_M,), jnp.int32),
            pltpu.VMEM((_N,), jnp.int32),
            pltpu.VMEM((_L * 16,), jnp.int32),
            pltpu.VMEM((_L * 32,), jnp.int32),
            pltpu.VMEM((_L * 16 + 16,), jnp.int32),
            pltpu.VMEM((_M,), jnp.int32),
            pltpu.VMEM((_L,), jnp.int32),
            pltpu.VMEM((_SPW,), jnp.int32),
        ],
    )
    qperm, spk, thist = csr(ql, tl)

    rowscale = pl.kernel(
        _rowscale_body,
        mesh=mesh,
        compiler_params=pltpu.CompilerParams(needs_layout_passes=False, use_tc_tiling_on_sc=False),
        out_type=jax.ShapeDtypeStruct((_M,), jnp.float32),
        scratch_types=[
            pltpu.VMEM((_T, _M // _NTILES), jnp.int32),
            pltpu.VMEM((_T, _L), jnp.int32),
            pltpu.VMEM((_M // _NTILES,), jnp.float32),
        ],
    )(ql, thist)

    qdata = pl.kernel(
        _qdata_body,
        mesh=mesh,
        compiler_params=pltpu.CompilerParams(needs_layout_passes=False, use_tc_tiling_on_sc=False),
        out_type=jax.ShapeDtypeStruct((_T, _M), jnp.int32),
        scratch_types=[
            pltpu.VMEM((_M,), jnp.int32),
            pltpu.VMEM((_M,), jnp.float32),
            pltpu.VMEM((_M,), jnp.int32),
        ],
    )(qperm, rowscale)

    out = pl.kernel(
        _join_body,
        mesh=mesh,
        compiler_params=pltpu.CompilerParams(needs_layout_passes=False, use_tc_tiling_on_sc=False),
        out_type=jax.ShapeDtypeStruct((_M, _N), jnp.float32),
        scratch_types=[
            pltpu.VMEM((_T, _M), jnp.int32),
            pltpu.VMEM((_T, _SPW), jnp.int32),
            pltpu.VMEM((2, _M, _W), jnp.float32),
            pltpu.VMEM((_T, _W), jnp.int32),
            pltpu.SemaphoreType.DMA((2,)),
            pltpu.SemaphoreType.DMA,
        ],
    )(tl, qdata, spk)
    return out


def kernel(X, query_leaves, train_leaves):
    del X
    return _sc_pipeline(query_leaves, train_leaves)


# speedup vs baseline: 8.0333x; 1.0591x over previous
import functools

import jax
import jax.numpy as jnp
from jax import lax
from jax.experimental import pallas as pl
from jax.experimental.pallas import tpu as pltpu
from jax.experimental.pallas import tpu_sc as plsc

_T = 64
_M = 1024
_N = 20000
_L = 512

_NTILES = 32
_TREES_PER_TILE = _T // _NTILES
_W = 16
_NSTRIPS = _N // _W
_SPW = 272


def _iota16():
    return lax.iota(jnp.int32, 16)


def _wid():
    return lax.axis_index("c") * 16 + lax.axis_index("s")


def _csr_body(ql_hbm, tl_hbm, qperm_hbm, spk_hbm, thist_hbm,
              qlv, tlv, cnt, cnt2, seg, out1k, out512, sbuf):
    w = _wid()
    lanes = _iota16()

    def do_tree(i, _):
        t = w * _TREES_PER_TILE + i
        pltpu.sync_copy(ql_hbm.at[t], qlv)
        pltpu.sync_copy(tl_hbm.at[t], tlv)

        z16 = jnp.zeros((16,), jnp.int32)

        def zero_blk(b, _):
            for u in range(8):
                cnt[pl.ds((b * 8 + u) * 16, 16)] = z16
            return 0
        lax.fori_loop(0, _L // 8, zero_blk, 0)

        def qhist(k, _):
            lv = plsc.load_gather(qlv, [lanes * (_M // 16) + k])
            a = lv * 16 + lanes
            c = plsc.load_gather(cnt, [a])
            plsc.store_scatter(cnt, [a], c + 1)
            return 0
        lax.fori_loop(0, _M // 16, qhist, 0)

        def scan_blk(b2, carry):
            b = b2 * 2
            v0 = cnt[pl.ds(b * 16, 16)]
            v1 = cnt[pl.ds(b * 16 + 16, 16)]
            inc0 = plsc.cumsum(v0)
            inc1 = plsc.cumsum(v1)
            carry0 = carry + inc0[15]
            seg[pl.ds(b * 16, 16)] = inc0 - v0 + carry
            seg[pl.ds(b * 16 + 16, 16)] = inc1 - v1 + carry0
            return carry0 + inc1[15]
        total = lax.fori_loop(0, _L // 2, scan_blk, jnp.int32(0))
        seg[pl.ds(_L * 16, 16)] = jnp.full((16,), total, jnp.int32)

        def spk_blk(b, _):
            wj = b * 16 + lanes
            e0 = jnp.minimum(wj * 2, _L) * 16
            e1 = jnp.minimum(wj * 2 + 1, _L) * 16
            lo = plsc.load_gather(seg, [e0])
            hi = plsc.load_gather(seg, [e1])
            sbuf[pl.ds(b * 16, 16)] = lo | lax.shift_left(hi, 16)
            return 0
        lax.fori_loop(0, _SPW // 16, spk_blk, 0)
        pltpu.sync_copy(sbuf, spk_hbm.at[t])

        def qscat(k, _):
            mv = lanes * (_M // 16) + k
            lv = plsc.load_gather(qlv, [mv])
            a = lv * 16 + lanes
            cur = plsc.load_gather(seg, [a])
            plsc.store_scatter(out1k, [cur], mv)
            plsc.store_scatter(seg, [a], cur + 1)
            return 0
        lax.fori_loop(0, _M // 16, qscat, 0)
        pltpu.sync_copy(out1k, qperm_hbm.at[t])

        def zero_blk2(b, _):
            for u in range(8):
                cnt2[pl.ds((b * 8 + u) * 16, 16)] = z16
            return 0
        lax.fori_loop(0, _L * 2 // 8, zero_blk2, 0)

        def thist_step(k2, _):
            k = k2 * 2
            lv0 = plsc.load_gather(tlv, [lanes * (_N // 16) + k])
            lv1 = plsc.load_gather(tlv, [lanes * (_N // 16) + k + 1])
            a0 = lv0 * 32 + lanes * 2
            a1 = lv1 * 32 + lanes * 2 + 1
            c0 = plsc.load_gather(cnt2, [a0])
            c1 = plsc.load_gather(cnt2, [a1])
            plsc.store_scatter(cnt2, [a0], c0 + 1)
            plsc.store_scatter(cnt2, [a1], c1 + 1)
            return 0
        lax.fori_loop(0, _N // 32, thist_step, 0)

        def tfold(b, _):
            l16 = b * 16 + lanes
            acc = plsc.load_gather(cnt2, [l16 * 32])
            def add_lane(j, acc):
                return acc + plsc.load_gather(cnt2, [l16 * 32 + j])
            acc = lax.fori_loop(1, 32, add_lane, acc)
            out512[pl.ds(b * 16, 16)] = acc
            return 0
        lax.fori_loop(0, _L // 16, tfold, 0)
        pltpu.sync_copy(out512, thist_hbm.at[t])
        return 0

    lax.fori_loop(0, _TREES_PER_TILE, do_tree, 0)


def _rowscale_body(ql_hbm, thist_hbm, rs_hbm, qlb, th, out32):
    w = _wid()
    mchunk = _M // _NTILES
    pltpu.sync_copy(thist_hbm, th)
    pltpu.sync_copy(ql_hbm.at[:, pl.ds(w * mchunk, mchunk)], qlb)

    def per_half(j):
        def per_tree(t, acc):
            lv = qlb[t, pl.ds(j * 16, 16)]
            h = plsc.load_gather(th, [jnp.full((16,), t, jnp.int32), lv])
            return acc + h
        acc = lax.fori_loop(0, _T, per_tree, jnp.zeros((16,), jnp.int32))
        denom = acc.astype(jnp.float32) + jnp.float32(_T * 1e-8)
        out32[pl.ds(j * 16, 16)] = jnp.float32(1.0) / denom

    per_half(0)
    per_half(1)
    pltpu.sync_copy(out32, rs_hbm.at[pl.ds(w * mchunk, mchunk)])


def _qdata_body(qperm_hbm, rs_hbm, qdata_hbm, qpv, rsv, qdv):
    w = _wid()
    pltpu.sync_copy(rs_hbm, rsv)

    def do_tree(i, _):
        t = w * _TREES_PER_TILE + i
        pltpu.sync_copy(qperm_hbm.at[t], qpv)

        def step(j, _):
            mv = qpv[pl.ds(j * 16, 16)]
            rb = plsc.bitcast(plsc.load_gather(rsv, [mv]), jnp.int32)
            rb = (rb + 0x8000) & jnp.int32(-65536)
            qdv[pl.ds(j * 16, 16)] = rb | mv
            return 0
        lax.fori_loop(0, _M // 16, step, 0)
        pltpu.sync_copy(qdv, qdata_hbm.at[t])
        return 0

    lax.fori_loop(0, _TREES_PER_TILE, do_tree, 0)


def _join_body(tl_hbm, qdata_hbm, spk_hbm, out_hbm,
               qdata, spk, acc, tlb, osem, tsem):
    w = _wid()
    lanes = _iota16()
    pltpu.sync_copy(qdata_hbm, qdata)
    pltpu.sync_copy(spk_hbm, spk)

    npass = (_NSTRIPS + _NTILES - 1) // _NTILES

    def strip_of(si):
        return w + si * _NTILES

    def tl_copy(si):
        return pltpu.make_async_copy(
            tl_hbm.at[:, pl.ds(strip_of(si) * _W, _W)], tlb, tsem)

    def out_copy(si, b):
        return pltpu.make_async_copy(
            acc.at[b], out_hbm.at[:, pl.ds(strip_of(si) * _W, _W)],
            osem.at[b])


    def do_pass(p, _):
        for h in range(2):
            si = p * 2 + h
            b = h

            @pl.when(strip_of(si) < _NSTRIPS)
            def _():
                acc_b = acc.at[b]

                tl_copy(si).start()

                @pl.when(si >= 2)
                def _():
                    out_copy(si - 2, b).wait()

                def zero_rows(r, _):
                    for rr in range(16):
                        acc_b[r * 16 + rr, :] = jnp.zeros((16,), jnp.float32)
                    return 0
                lax.fori_loop(0, _M // 16, zero_rows, 0)
                tl_copy(si).wait()

                def header(t):
                    tlv = tlb[t, :]
                    trow = jnp.full((16,), t, jnp.int32)
                    wv = lax.shift_right_logical(tlv, 1)
                    odd = (tlv & 1) > 0
                    u0 = plsc.load_gather(spk, [trow, wv])
                    u1 = plsc.load_gather(spk, [trow, wv + 1])
                    lo0 = u0 & 0xFFFF
                    hi0 = lax.shift_right_logical(u0, 16)
                    lo1 = u1 & 0xFFFF
                    start = jnp.where(odd, hi0, lo0)
                    ln = jnp.where(odd, lo1, hi0) - start
                    return trow, start, ln

                def expand_tree(trow, start, ln):
                    mx = jnp.max(ln)

                    def expand(k2, _):
                        k = k2 * 2
                        msk0 = ln > k
                        msk1 = ln > k + 1
                        idx0 = jnp.where(msk0, start + k, 0)
                        idx1 = jnp.where(msk1, start + k + 1, 0)
                        q0 = plsc.load_gather(qdata, [trow, idx0])
                        q1 = plsc.load_gather(qdata, [trow, idx1])
                        mm0 = q0 & 0xFFFF
                        mm1 = q1 & 0xFFFF
                        v0 = plsc.bitcast(q0 & jnp.int32(-65536), jnp.float32)
                        v1 = plsc.bitcast(q1 & jnp.int32(-65536), jnp.float32)
                        plsc.addupdate_scatter(
                            acc_b, [mm0, lanes], v0, mask=msk0)
                        plsc.addupdate_scatter(
                            acc_b, [mm1, lanes], v1, mask=msk1)
                        return 0
                    lax.fori_loop(0, (mx + 1) // 2, expand, 0)

                def per_pair(tp, _):
                    h0 = header(tp * 2)
                    h1 = header(tp * 2 + 1)
                    expand_tree(*h0)
                    expand_tree(*h1)
                    return 0
                lax.fori_loop(0, _T // 2, per_pair, 0)
                out_copy(si, b).start()
        return 0

    lax.fori_loop(0, (npass + 1) // 2, do_pass, 0)

    nstrips = _NSTRIPS // _NTILES + jnp.where(w < _NSTRIPS % _NTILES, 1, 0)

    @pl.when(nstrips >= 2)
    def _():
        out_copy(nstrips - 2, (nstrips - 2) % 2).wait()

    @pl.when(nstrips >= 1)
    def _():
        out_copy(nstrips - 1, (nstrips - 1) % 2).wait()


def _sc_pipeline(ql, tl):
    mesh = plsc.VectorSubcoreMesh(core_axis_name="c", subcore_axis_name="s")

    csr = pl.kernel(
        _csr_body,
        mesh=mesh,
        compiler_params=pltpu.CompilerParams(needs_layout_passes=False, use_tc_tiling_on_sc=False),
        out_type=(
            jax.ShapeDtypeStruct((_T, _M), jnp.int32),
            jax.ShapeDtypeStruct((_T, _SPW), jnp.int32),
            jax.ShapeDtypeStruct((_T, _L), jnp.int32),
        ),
        scratch_types=[
            pltpu.VMEM((_M,), jnp.int32),
            pltpu.VMEM((_N,), jnp.int32),
            pltpu.VMEM((_L * 16,), jnp.int32),
            pltpu.VMEM((_L * 32,), jnp.int32),
            pltpu.VMEM((_L * 16 + 16,), jnp.int32),
            pltpu.VMEM((_M,), jnp.int32),
            pltpu.VMEM((_L,), jnp.int32),
            pltpu.VMEM((_SPW,), jnp.int32),
        ],
    )
    qperm, spk, thist = csr(ql, tl)

    rowscale = pl.kernel(
        _rowscale_body,
        mesh=mesh,
        compiler_params=pltpu.CompilerParams(needs_layout_passes=False, use_tc_tiling_on_sc=False),
        out_type=jax.ShapeDtypeStruct((_M,), jnp.float32),
        scratch_types=[
            pltpu.VMEM((_T, _M // _NTILES), jnp.int32),
            pltpu.VMEM((_T, _L), jnp.int32),
            pltpu.VMEM((_M // _NTILES,), jnp.float32),
        ],
    )(ql, thist)

    qdata = pl.kernel(
        _qdata_body,
        mesh=mesh,
        compiler_params=pltpu.CompilerParams(needs_layout_passes=False, use_tc_tiling_on_sc=False),
        out_type=jax.ShapeDtypeStruct((_T, _M), jnp.int32),
        scratch_types=[
            pltpu.VMEM((_M,), jnp.int32),
            pltpu.VMEM((_M,), jnp.float32),
            pltpu.VMEM((_M,), jnp.int32),
        ],
    )(qperm, rowscale)

    out = pl.kernel(
        _join_body,
        mesh=mesh,
        compiler_params=pltpu.CompilerParams(needs_layout_passes=False, use_tc_tiling_on_sc=False),
        out_type=jax.ShapeDtypeStruct((_M, _N), jnp.float32),
        scratch_types=[
            pltpu.VMEM((_T, _M), jnp.int32),
            pltpu.VMEM((_T, _SPW), jnp.int32),
            pltpu.VMEM((2, _M, _W), jnp.float32),
            pltpu.VMEM((_T, _W), jnp.int32),
            pltpu.SemaphoreType.DMA((2,)),
            pltpu.SemaphoreType.DMA,
        ],
    )(tl, qdata, spk)
    return out


def kernel(X, query_leaves, train_leaves):
    del X
    return _sc_pipeline(query_leaves, train_leaves)


# speedup vs baseline: 8.1164x; 1.0103x over previous
import functools

import jax
import jax.numpy as jnp
from jax import lax
from jax.experimental import pallas as pl
from jax.experimental.pallas import tpu as pltpu
from jax.experimental.pallas import tpu_sc as plsc

_T = 64
_M = 1024
_N = 20000
_L = 512

_NTILES = 32
_TREES_PER_TILE = _T // _NTILES
_W = 16
_NSTRIPS = _N // _W
_SPW = 272


def _iota16():
    return lax.iota(jnp.int32, 16)


def _wid():
    return lax.axis_index("c") * 16 + lax.axis_index("s")


def _csr_body(ql_hbm, tl_hbm, qperm_hbm, spk_hbm, thist_hbm,
              qlv, tlv, cnt, cnt2, seg, out1k, out512, sbuf):
    w = _wid()
    lanes = _iota16()

    def do_tree(i, _):
        t = w * _TREES_PER_TILE + i
        pltpu.sync_copy(ql_hbm.at[t], qlv)
        pltpu.sync_copy(tl_hbm.at[t], tlv)

        z16 = jnp.zeros((16,), jnp.int32)

        def zero_blk(b, _):
            for u in range(8):
                cnt[pl.ds((b * 8 + u) * 16, 16)] = z16
            return 0
        lax.fori_loop(0, _L // 8, zero_blk, 0)

        def qhist(k, _):
            lv = plsc.load_gather(qlv, [lanes * (_M // 16) + k])
            a = lv * 16 + lanes
            c = plsc.load_gather(cnt, [a])
            plsc.store_scatter(cnt, [a], c + 1)
            return 0
        lax.fori_loop(0, _M // 16, qhist, 0)

        def scan_blk(b2, carry):
            b = b2 * 2
            v0 = cnt[pl.ds(b * 16, 16)]
            v1 = cnt[pl.ds(b * 16 + 16, 16)]
            inc0 = plsc.cumsum(v0)
            inc1 = plsc.cumsum(v1)
            carry0 = carry + inc0[15]
            seg[pl.ds(b * 16, 16)] = inc0 - v0 + carry
            seg[pl.ds(b * 16 + 16, 16)] = inc1 - v1 + carry0
            return carry0 + inc1[15]
        total = lax.fori_loop(0, _L // 2, scan_blk, jnp.int32(0))
        seg[pl.ds(_L * 16, 16)] = jnp.full((16,), total, jnp.int32)

        def spk_blk(b, _):
            wj = b * 16 + lanes
            e0 = jnp.minimum(wj * 2, _L) * 16
            e1 = jnp.minimum(wj * 2 + 1, _L) * 16
            lo = plsc.load_gather(seg, [e0])
            hi = plsc.load_gather(seg, [e1])
            sbuf[pl.ds(b * 16, 16)] = lo | lax.shift_left(hi, 16)
            return 0
        lax.fori_loop(0, _SPW // 16, spk_blk, 0)
        pltpu.sync_copy(sbuf, spk_hbm.at[t])

        def qscat(k, _):
            mv = lanes * (_M // 16) + k
            lv = plsc.load_gather(qlv, [mv])
            a = lv * 16 + lanes
            cur = plsc.load_gather(seg, [a])
            plsc.store_scatter(out1k, [cur], mv)
            plsc.store_scatter(seg, [a], cur + 1)
            return 0
        lax.fori_loop(0, _M // 16, qscat, 0)
        pltpu.sync_copy(out1k, qperm_hbm.at[t])

        def zero_blk2(b, _):
            for u in range(8):
                cnt2[pl.ds((b * 8 + u) * 16, 16)] = z16
            return 0
        lax.fori_loop(0, _L * 2 // 8, zero_blk2, 0)

        def thist_step(k2, _):
            k = k2 * 2
            lv0 = plsc.load_gather(tlv, [lanes * (_N // 16) + k])
            lv1 = plsc.load_gather(tlv, [lanes * (_N // 16) + k + 1])
            a0 = lv0 * 32 + lanes * 2
            a1 = lv1 * 32 + lanes * 2 + 1
            c0 = plsc.load_gather(cnt2, [a0])
            c1 = plsc.load_gather(cnt2, [a1])
            plsc.store_scatter(cnt2, [a0], c0 + 1)
            plsc.store_scatter(cnt2, [a1], c1 + 1)
            return 0
        lax.fori_loop(0, _N // 32, thist_step, 0)

        def tfold(b, _):
            l16 = b * 16 + lanes
            acc = plsc.load_gather(cnt2, [l16 * 32])
            def add_lane(j, acc):
                return acc + plsc.load_gather(cnt2, [l16 * 32 + j])
            acc = lax.fori_loop(1, 32, add_lane, acc)
            out512[pl.ds(b * 16, 16)] = acc
            return 0
        lax.fori_loop(0, _L // 16, tfold, 0)
        pltpu.sync_copy(out512, thist_hbm.at[t])
        return 0

    lax.fori_loop(0, _TREES_PER_TILE, do_tree, 0)


def _rowscale_body(ql_hbm, thist_hbm, rs_hbm, qlb, th, out32):
    w = _wid()
    mchunk = _M // _NTILES
    pltpu.sync_copy(thist_hbm, th)
    pltpu.sync_copy(ql_hbm.at[:, pl.ds(w * mchunk, mchunk)], qlb)

    def per_half(j):
        def per_tree(t, acc):
            lv = qlb[t, pl.ds(j * 16, 16)]
            h = plsc.load_gather(th, [jnp.full((16,), t, jnp.int32), lv])
            return acc + h
        acc = lax.fori_loop(0, _T, per_tree, jnp.zeros((16,), jnp.int32))
        denom = acc.astype(jnp.float32) + jnp.float32(_T * 1e-8)
        out32[pl.ds(j * 16, 16)] = jnp.float32(1.0) / denom

    per_half(0)
    per_half(1)
    pltpu.sync_copy(out32, rs_hbm.at[pl.ds(w * mchunk, mchunk)])


def _qdata_body(qperm_hbm, rs_hbm, qdata_hbm, qpv, rsv, qdv):
    w = _wid()
    pltpu.sync_copy(rs_hbm, rsv)

    def do_tree(i, _):
        t = w * _TREES_PER_TILE + i
        pltpu.sync_copy(qperm_hbm.at[t], qpv)

        def step(j, _):
            mv = qpv[pl.ds(j * 16, 16)]
            rb = plsc.bitcast(plsc.load_gather(rsv, [mv]), jnp.int32)
            rb = (rb + 0x8000) & jnp.int32(-65536)
            qdv[pl.ds(j * 16, 16)] = rb | mv
            return 0
        lax.fori_loop(0, _M // 16, step, 0)
        pltpu.sync_copy(qdv, qdata_hbm.at[t])
        return 0

    lax.fori_loop(0, _TREES_PER_TILE, do_tree, 0)


def _join_body(ql_hbm, tl_hbm, qperm_hbm, spk_hbm, thist_hbm,
               out_hbm, rs_stage, qdata_stage,
               qdata, spk, acc, tlb, qlb, qpv, rsl, rsv, osem, tsem):
    c = lax.axis_index("c")
    sidx = lax.axis_index("s")
    w = c * 16 + sidx
    lanes = _iota16()
    pltpu.sync_copy(spk_hbm, spk)

    pltpu.sync_copy(thist_hbm, qdata.at[:, pl.ds(0, _L)])
    pltpu.sync_copy(ql_hbm.at[:, pl.ds(sidx * 64, 64)], qlb)

    for j in range(4):
        def per_tree_rs(t, accv):
            lv = qlb[t, pl.ds(j * 16, 16)]
            h = plsc.load_gather(qdata, [jnp.full((16,), t, jnp.int32), lv])
            return accv + h
        accv = lax.fori_loop(0, _T, per_tree_rs, jnp.zeros((16,), jnp.int32))
        denom = accv.astype(jnp.float32) + jnp.float32(_T * 1e-8)
        rsl[pl.ds(j * 16, 16)] = jnp.float32(1.0) / denom
    pltpu.sync_copy(rsl, rs_stage.at[c, pl.ds(sidx * 64, 64)])
    plsc.subcore_barrier()

    pltpu.sync_copy(rs_stage.at[c], rsv)
    pltpu.sync_copy(qperm_hbm.at[pl.ds(sidx * 4, 4)], qpv)
    for i in range(4):
        def pack_step(j, _):
            mv = qpv[i, pl.ds(j * 16, 16)]
            rb = plsc.bitcast(plsc.load_gather(rsv, [mv]), jnp.int32)
            rb = (rb + 0x8000) & jnp.int32(-65536)
            qpv[i, pl.ds(j * 16, 16)] = rb | mv
            return 0
        lax.fori_loop(0, _M // 16, pack_step, 0)
    pltpu.sync_copy(qpv, qdata_stage.at[c, pl.ds(sidx * 4, 4)])
    plsc.subcore_barrier()

    pltpu.sync_copy(qdata_stage.at[c], qdata)

    npass = (_NSTRIPS + _NTILES - 1) // _NTILES

    def strip_of(si):
        return w + si * _NTILES

    def tl_copy(si):
        return pltpu.make_async_copy(
            tl_hbm.at[:, pl.ds(strip_of(si) * _W, _W)], tlb, tsem)

    def out_copy(si, b):
        return pltpu.make_async_copy(
            acc.at[b], out_hbm.at[:, pl.ds(strip_of(si) * _W, _W)],
            osem.at[b])


    def do_pass(p, _):
        for h in range(2):
            si = p * 2 + h
            b = h

            @pl.when(strip_of(si) < _NSTRIPS)
            def _():
                acc_b = acc.at[b]

                tl_copy(si).start()

                @pl.when(si >= 2)
                def _():
                    out_copy(si - 2, b).wait()

                def zero_rows(r, _):
                    for rr in range(16):
                        acc_b[r * 16 + rr, :] = jnp.zeros((16,), jnp.float32)
                    return 0
                lax.fori_loop(0, _M // 16, zero_rows, 0)
                tl_copy(si).wait()

                def header(t):
                    tlv = tlb[t, :]
                    trow = jnp.full((16,), t, jnp.int32)
                    wv = lax.shift_right_logical(tlv, 1)
                    odd = (tlv & 1) > 0
                    u0 = plsc.load_gather(spk, [trow, wv])
                    u1 = plsc.load_gather(spk, [trow, wv + 1])
                    lo0 = u0 & 0xFFFF
                    hi0 = lax.shift_right_logical(u0, 16)
                    lo1 = u1 & 0xFFFF
                    start = jnp.where(odd, hi0, lo0)
                    ln = jnp.where(odd, lo1, hi0) - start
                    return trow, start, ln

                def expand_tree(trow, start, ln):
                    mx = jnp.max(ln)

                    def expand(k2, _):
                        k = k2 * 2
                        msk0 = ln > k
                        msk1 = ln > k + 1
                        idx0 = jnp.where(msk0, start + k, 0)
                        idx1 = jnp.where(msk1, start + k + 1, 0)
                        q0 = plsc.load_gather(qdata, [trow, idx0])
                        q1 = plsc.load_gather(qdata, [trow, idx1])
                        mm0 = q0 & 0xFFFF
                        mm1 = q1 & 0xFFFF
                        v0 = plsc.bitcast(q0 & jnp.int32(-65536), jnp.float32)
                        v1 = plsc.bitcast(q1 & jnp.int32(-65536), jnp.float32)
                        plsc.addupdate_scatter(
                            acc_b, [mm0, lanes], v0, mask=msk0)
                        plsc.addupdate_scatter(
                            acc_b, [mm1, lanes], v1, mask=msk1)
                        return 0
                    lax.fori_loop(0, (mx + 1) // 2, expand, 0)

                def per_pair(tp, _):
                    h0 = header(tp * 2)
                    h1 = header(tp * 2 + 1)
                    expand_tree(*h0)
                    expand_tree(*h1)
                    return 0
                lax.fori_loop(0, _T // 2, per_pair, 0)
                out_copy(si, b).start()
        return 0

    lax.fori_loop(0, (npass + 1) // 2, do_pass, 0)

    nstrips = _NSTRIPS // _NTILES + jnp.where(w < _NSTRIPS % _NTILES, 1, 0)

    @pl.when(nstrips >= 2)
    def _():
        out_copy(nstrips - 2, (nstrips - 2) % 2).wait()

    @pl.when(nstrips >= 1)
    def _():
        out_copy(nstrips - 1, (nstrips - 1) % 2).wait()


def _sc_pipeline(ql, tl):
    mesh = plsc.VectorSubcoreMesh(core_axis_name="c", subcore_axis_name="s")

    csr = pl.kernel(
        _csr_body,
        mesh=mesh,
        compiler_params=pltpu.CompilerParams(needs_layout_passes=False, use_tc_tiling_on_sc=False),
        out_type=(
            jax.ShapeDtypeStruct((_T, _M), jnp.int32),
            jax.ShapeDtypeStruct((_T, _SPW), jnp.int32),
            jax.ShapeDtypeStruct((_T, _L), jnp.int32),
        ),
        scratch_types=[
            pltpu.VMEM((_M,), jnp.int32),
            pltpu.VMEM((_N,), jnp.int32),
            pltpu.VMEM((_L * 16,), jnp.int32),
            pltpu.VMEM((_L * 32,), jnp.int32),
            pltpu.VMEM((_L * 16 + 16,), jnp.int32),
            pltpu.VMEM((_M,), jnp.int32),
            pltpu.VMEM((_L,), jnp.int32),
            pltpu.VMEM((_SPW,), jnp.int32),
        ],
    )
    qperm, spk, thist = csr(ql, tl)

    out, _, _ = pl.kernel(
        _join_body,
        mesh=mesh,
        compiler_params=pltpu.CompilerParams(needs_layout_passes=False, use_tc_tiling_on_sc=False),
        out_type=(
            jax.ShapeDtypeStruct((_M, _N), jnp.float32),
            jax.ShapeDtypeStruct((2, _M), jnp.float32),
            jax.ShapeDtypeStruct((2, _T, _M), jnp.int32),
        ),
        scratch_types=[
            pltpu.VMEM((_T, _M), jnp.int32),
            pltpu.VMEM((_T, _SPW), jnp.int32),
            pltpu.VMEM((2, _M, _W), jnp.float32),
            pltpu.VMEM((_T, _W), jnp.int32),
            pltpu.VMEM((_T, 64), jnp.int32),
            pltpu.VMEM((4, _M), jnp.int32),
            pltpu.VMEM((64,), jnp.float32),
            pltpu.VMEM((_M,), jnp.float32),
            pltpu.SemaphoreType.DMA((2,)),
            pltpu.SemaphoreType.DMA,
        ],
    )(ql, tl, qperm, spk, thist)
    return out


def kernel(X, query_leaves, train_leaves):
    del X
    return _sc_pipeline(query_leaves, train_leaves)


# speedup vs baseline: 9.4651x; 1.1662x over previous
import functools

import jax
import jax.numpy as jnp
from jax import lax
from jax.experimental import pallas as pl
from jax.experimental.pallas import tpu as pltpu
from jax.experimental.pallas import tpu_sc as plsc

_T = 64
_M = 1024
_N = 20000
_L = 512

_NTILES = 32
_TREES_PER_TILE = _T // _NTILES
_W = 16
_NSTRIPS = _N // _W
_SPW = 272


def _iota16():
    return lax.iota(jnp.int32, 16)


def _wid():
    return lax.axis_index("c") * 16 + lax.axis_index("s")


def _csr_body(ql_hbm, tl_hbm, qperm_hbm, spk_hbm, thist_hbm,
              qlv, tlv, cnt, cnt2, seg, out1k, out512, sbuf):
    w = _wid()
    lanes = _iota16()

    def do_tree(i, _):
        t = w * _TREES_PER_TILE + i
        pltpu.sync_copy(ql_hbm.at[t], qlv)
        pltpu.sync_copy(tl_hbm.at[t], tlv)

        z16 = jnp.zeros((16,), jnp.int32)

        def zero_blk(b, _):
            for u in range(8):
                cnt[pl.ds((b * 8 + u) * 16, 16)] = z16
            return 0
        lax.fori_loop(0, _L // 8, zero_blk, 0)

        def qhist(k, _):
            lv = plsc.load_gather(qlv, [lanes * (_M // 16) + k])
            a = lv * 16 + lanes
            c = plsc.load_gather(cnt, [a])
            plsc.store_scatter(cnt, [a], c + 1)
            return 0
        lax.fori_loop(0, _M // 16, qhist, 0)

        def scan_blk(b2, carry):
            b = b2 * 2
            v0 = cnt[pl.ds(b * 16, 16)]
            v1 = cnt[pl.ds(b * 16 + 16, 16)]
            inc0 = plsc.cumsum(v0)
            inc1 = plsc.cumsum(v1)
            carry0 = carry + inc0[15]
            seg[pl.ds(b * 16, 16)] = inc0 - v0 + carry
            seg[pl.ds(b * 16 + 16, 16)] = inc1 - v1 + carry0
            return carry0 + inc1[15]
        total = lax.fori_loop(0, _L // 2, scan_blk, jnp.int32(0))
        seg[pl.ds(_L * 16, 16)] = jnp.full((16,), total, jnp.int32)

        def spk_blk(b, _):
            wj = b * 16 + lanes
            e0 = jnp.minimum(wj * 2, _L) * 16
            e1 = jnp.minimum(wj * 2 + 1, _L) * 16
            lo = plsc.load_gather(seg, [e0])
            hi = plsc.load_gather(seg, [e1])
            sbuf[pl.ds(b * 16, 16)] = lo | lax.shift_left(hi, 16)
            return 0
        lax.fori_loop(0, _SPW // 16, spk_blk, 0)
        pltpu.sync_copy(sbuf, spk_hbm.at[t])

        def qscat(k, _):
            mv = lanes * (_M // 16) + k
            lv = plsc.load_gather(qlv, [mv])
            a = lv * 16 + lanes
            cur = plsc.load_gather(seg, [a])
            plsc.store_scatter(out1k, [cur], mv)
            plsc.store_scatter(seg, [a], cur + 1)
            return 0
        lax.fori_loop(0, _M // 16, qscat, 0)
        pltpu.sync_copy(out1k, qperm_hbm.at[t])

        def zero_blk2(b, _):
            for u in range(8):
                cnt2[pl.ds((b * 8 + u) * 16, 16)] = z16
            return 0
        lax.fori_loop(0, _L * 2 // 8, zero_blk2, 0)

        def thist_step(k2, _):
            k = k2 * 2
            lv0 = plsc.load_gather(tlv, [lanes * (_N // 16) + k])
            lv1 = plsc.load_gather(tlv, [lanes * (_N // 16) + k + 1])
            a0 = lv0 * 32 + lanes * 2
            a1 = lv1 * 32 + lanes * 2 + 1
            c0 = plsc.load_gather(cnt2, [a0])
            c1 = plsc.load_gather(cnt2, [a1])
            plsc.store_scatter(cnt2, [a0], c0 + 1)
            plsc.store_scatter(cnt2, [a1], c1 + 1)
            return 0
        lax.fori_loop(0, _N // 32, thist_step, 0)

        def tfold(b, _):
            l16 = b * 16 + lanes
            acc = plsc.load_gather(cnt2, [l16 * 32])
            def add_lane(j, acc):
                return acc + plsc.load_gather(cnt2, [l16 * 32 + j])
            acc = lax.fori_loop(1, 32, add_lane, acc)
            out512[pl.ds(b * 16, 16)] = acc
            return 0
        lax.fori_loop(0, _L // 16, tfold, 0)
        pltpu.sync_copy(out512, thist_hbm.at[t])
        return 0

    lax.fori_loop(0, _TREES_PER_TILE, do_tree, 0)


def _rowscale_body(ql_hbm, thist_hbm, rs_hbm, qlb, th, out32):
    w = _wid()
    mchunk = _M // _NTILES
    pltpu.sync_copy(thist_hbm, th)
    pltpu.sync_copy(ql_hbm.at[:, pl.ds(w * mchunk, mchunk)], qlb)

    def per_half(j):
        def per_tree(t, acc):
            lv = qlb[t, pl.ds(j * 16, 16)]
            h = plsc.load_gather(th, [jnp.full((16,), t, jnp.int32), lv])
            return acc + h
        acc = lax.fori_loop(0, _T, per_tree, jnp.zeros((16,), jnp.int32))
        denom = acc.astype(jnp.float32) + jnp.float32(_T * 1e-8)
        out32[pl.ds(j * 16, 16)] = jnp.float32(1.0) / denom

    per_half(0)
    per_half(1)
    pltpu.sync_copy(out32, rs_hbm.at[pl.ds(w * mchunk, mchunk)])


def _qdata_body(qperm_hbm, rs_hbm, qdata_hbm, qpv, rsv, qdv):
    w = _wid()
    pltpu.sync_copy(rs_hbm, rsv)

    def do_tree(i, _):
        t = w * _TREES_PER_TILE + i
        pltpu.sync_copy(qperm_hbm.at[t], qpv)

        def step(j, _):
            mv = qpv[pl.ds(j * 16, 16)]
            rb = plsc.bitcast(plsc.load_gather(rsv, [mv]), jnp.int32)
            rb = (rb + 0x8000) & jnp.int32(-65536)
            qdv[pl.ds(j * 16, 16)] = rb | mv
            return 0
        lax.fori_loop(0, _M // 16, step, 0)
        pltpu.sync_copy(qdv, qdata_hbm.at[t])
        return 0

    lax.fori_loop(0, _TREES_PER_TILE, do_tree, 0)


def _join_body(ql_hbm, tl_hbm, qperm_hbm, spk_hbm, thist_hbm,
               out_hbm, rs_stage, qdata_stage,
               qdata, spk, acc, tlb, qlb, qpv, rsl, rsv, osem, tsem):
    c = lax.axis_index("c")
    sidx = lax.axis_index("s")
    w = c * 16 + sidx
    lanes = _iota16()
    pltpu.sync_copy(spk_hbm, spk)

    pltpu.sync_copy(thist_hbm, qdata.at[:, pl.ds(0, _L)])
    pltpu.sync_copy(ql_hbm.at[:, pl.ds(sidx * 64, 64)], qlb)

    for j in range(4):
        def per_tree_rs(t, accv):
            lv = qlb[t, pl.ds(j * 16, 16)]
            h = plsc.load_gather(qdata, [jnp.full((16,), t, jnp.int32), lv])
            return accv + h
        accv = lax.fori_loop(0, _T, per_tree_rs, jnp.zeros((16,), jnp.int32))
        denom = accv.astype(jnp.float32) + jnp.float32(_T * 1e-8)
        rsl[pl.ds(j * 16, 16)] = jnp.float32(1.0) / denom
    pltpu.sync_copy(rsl, rs_stage.at[c, pl.ds(sidx * 64, 64)])
    plsc.subcore_barrier()

    pltpu.sync_copy(rs_stage.at[c], rsv)
    pltpu.sync_copy(qperm_hbm.at[pl.ds(sidx * 4, 4)], qpv)
    for i in range(4):
        def pack_step(j, _):
            mv = qpv[i, pl.ds(j * 16, 16)]
            rb = plsc.bitcast(plsc.load_gather(rsv, [mv]), jnp.int32)
            rb = (rb + 0x8000) & jnp.int32(-65536)
            qpv[i, pl.ds(j * 16, 16)] = rb | mv
            return 0
        lax.fori_loop(0, _M // 16, pack_step, 0)
    pltpu.sync_copy(qpv, qdata_stage.at[c, pl.ds(sidx * 4, 4)])
    plsc.subcore_barrier()

    pltpu.sync_copy(qdata_stage.at[c], qdata)

    npass = (_NSTRIPS + _NTILES - 1) // _NTILES

    def strip_of(si):
        return w + si * _NTILES

    def tl_copy(si):
        return pltpu.make_async_copy(
            tl_hbm.at[:, pl.ds(strip_of(si) * _W, _W)], tlb, tsem)

    def out_copy(si, b):
        return pltpu.make_async_copy(
            acc.at[b], out_hbm.at[:, pl.ds(strip_of(si) * _W, _W)],
            osem.at[b])


    def do_pass(p, _):
        for h in range(2):
            si = p * 2 + h
            b = h

            @pl.when(strip_of(si) < _NSTRIPS)
            def _():
                acc_b = acc.at[b]

                tl_copy(si).start()

                @pl.when(si >= 2)
                def _():
                    out_copy(si - 2, b).wait()

                def zero_rows(r, _):
                    for rr in range(16):
                        acc_b[r * 16 + rr, :] = jnp.zeros((16,), jnp.float32)
                    return 0
                lax.fori_loop(0, _M // 16, zero_rows, 0)
                tl_copy(si).wait()

                def header(t):
                    tlv = tlb[t, :]
                    trow = jnp.full((16,), t, jnp.int32)
                    wv = lax.shift_right_logical(tlv, 1)
                    odd = (tlv & 1) > 0
                    u0 = plsc.load_gather(spk, [trow, wv])
                    u1 = plsc.load_gather(spk, [trow, wv + 1])
                    lo0 = u0 & 0xFFFF
                    hi0 = lax.shift_right_logical(u0, 16)
                    lo1 = u1 & 0xFFFF
                    start = jnp.where(odd, hi0, lo0)
                    ln = jnp.where(odd, lo1, hi0) - start
                    return trow, start, ln

                def per_pair(tp, _):
                    trow0, start0, ln0 = header(tp * 2)
                    trow1, start1, ln1 = header(tp * 2 + 1)
                    mx = jnp.max(jnp.maximum(ln0, ln1))

                    def expand(k2, _):
                        k = k2 * 2
                        mskA = ln0 > k
                        mskB = ln0 > k + 1
                        mskC = ln1 > k
                        mskD = ln1 > k + 1
                        qA = plsc.load_gather(qdata, [trow0, start0 + k])
                        qB = plsc.load_gather(qdata, [trow0, start0 + k + 1])
                        qC = plsc.load_gather(qdata, [trow1, start1 + k])
                        qD = plsc.load_gather(qdata, [trow1, start1 + k + 1])
                        for q, msk in ((qA, mskA), (qB, mskB),
                                       (qC, mskC), (qD, mskD)):
                            plsc.addupdate_scatter(
                                acc_b, [q & 0xFFFF, lanes],
                                plsc.bitcast(q & jnp.int32(-65536),
                                             jnp.float32),
                                mask=msk)
                        return 0
                    lax.fori_loop(0, (mx + 1) // 2, expand, 0)
                    return 0
                lax.fori_loop(0, _T // 2, per_pair, 0)
                out_copy(si, b).start()
        return 0

    lax.fori_loop(0, (npass + 1) // 2, do_pass, 0)

    nstrips = _NSTRIPS // _NTILES + jnp.where(w < _NSTRIPS % _NTILES, 1, 0)

    @pl.when(nstrips >= 2)
    def _():
        out_copy(nstrips - 2, (nstrips - 2) % 2).wait()

    @pl.when(nstrips >= 1)
    def _():
        out_copy(nstrips - 1, (nstrips - 1) % 2).wait()


def _sc_pipeline(ql, tl):
    mesh = plsc.VectorSubcoreMesh(core_axis_name="c", subcore_axis_name="s")

    csr = pl.kernel(
        _csr_body,
        mesh=mesh,
        compiler_params=pltpu.CompilerParams(needs_layout_passes=False, use_tc_tiling_on_sc=False),
        out_type=(
            jax.ShapeDtypeStruct((_T, _M), jnp.int32),
            jax.ShapeDtypeStruct((_T, _SPW), jnp.int32),
            jax.ShapeDtypeStruct((_T, _L), jnp.int32),
        ),
        scratch_types=[
            pltpu.VMEM((_M,), jnp.int32),
            pltpu.VMEM((_N,), jnp.int32),
            pltpu.VMEM((_L * 16,), jnp.int32),
            pltpu.VMEM((_L * 32,), jnp.int32),
            pltpu.VMEM((_L * 16 + 16,), jnp.int32),
            pltpu.VMEM((_M,), jnp.int32),
            pltpu.VMEM((_L,), jnp.int32),
            pltpu.VMEM((_SPW,), jnp.int32),
        ],
    )
    qperm, spk, thist = csr(ql, tl)

    out, _, _ = pl.kernel(
        _join_body,
        mesh=mesh,
        compiler_params=pltpu.CompilerParams(needs_layout_passes=False, use_tc_tiling_on_sc=False),
        out_type=(
            jax.ShapeDtypeStruct((_M, _N), jnp.float32),
            jax.ShapeDtypeStruct((2, _M), jnp.float32),
            jax.ShapeDtypeStruct((2, _T, _M), jnp.int32),
        ),
        scratch_types=[
            pltpu.VMEM((_T, _M), jnp.int32),
            pltpu.VMEM((_T, _SPW), jnp.int32),
            pltpu.VMEM((2, _M, _W), jnp.float32),
            pltpu.VMEM((_T, _W), jnp.int32),
            pltpu.VMEM((_T, 64), jnp.int32),
            pltpu.VMEM((4, _M), jnp.int32),
            pltpu.VMEM((64,), jnp.float32),
            pltpu.VMEM((_M,), jnp.float32),
            pltpu.SemaphoreType.DMA((2,)),
            pltpu.SemaphoreType.DMA,
        ],
    )(ql, tl, qperm, spk, thist)
    return out


def kernel(X, query_leaves, train_leaves):
    del X
    return _sc_pipeline(query_leaves, train_leaves)


# speedup vs baseline: 10.3097x; 1.0892x over previous
import functools

import jax
import jax.numpy as jnp
from jax import lax
from jax.experimental import pallas as pl
from jax.experimental.pallas import tpu as pltpu
from jax.experimental.pallas import tpu_sc as plsc

_T = 64
_M = 1024
_N = 20000
_L = 512

_NTILES = 32
_TREES_PER_TILE = _T // _NTILES
_W = 16
_NSTRIPS = _N // _W
_SPW = 272


def _iota16():
    return lax.iota(jnp.int32, 16)


def _wid():
    return lax.axis_index("c") * 16 + lax.axis_index("s")


def _csr_body(ql_hbm, tl_hbm, qperm_hbm, spk_hbm, thist_hbm,
              qlv, tlv, cnt, cnt2, seg, out1k, out512, sbuf):
    w = _wid()
    lanes = _iota16()

    def do_tree(i, _):
        t = w * _TREES_PER_TILE + i
        pltpu.sync_copy(ql_hbm.at[t], qlv)
        pltpu.sync_copy(tl_hbm.at[t], tlv)

        z16 = jnp.zeros((16,), jnp.int32)

        def zero_blk(b, _):
            for u in range(8):
                cnt[pl.ds((b * 8 + u) * 16, 16)] = z16
            return 0
        lax.fori_loop(0, _L // 8, zero_blk, 0)

        def qhist(k, _):
            lv = plsc.load_gather(qlv, [lanes * (_M // 16) + k])
            a = lv * 16 + lanes
            c = plsc.load_gather(cnt, [a])
            plsc.store_scatter(cnt, [a], c + 1)
            return 0
        lax.fori_loop(0, _M // 16, qhist, 0)

        def scan_blk(b2, carry):
            b = b2 * 2
            v0 = cnt[pl.ds(b * 16, 16)]
            v1 = cnt[pl.ds(b * 16 + 16, 16)]
            inc0 = plsc.cumsum(v0)
            inc1 = plsc.cumsum(v1)
            carry0 = carry + inc0[15]
            seg[pl.ds(b * 16, 16)] = inc0 - v0 + carry
            seg[pl.ds(b * 16 + 16, 16)] = inc1 - v1 + carry0
            return carry0 + inc1[15]
        total = lax.fori_loop(0, _L // 2, scan_blk, jnp.int32(0))
        seg[pl.ds(_L * 16, 16)] = jnp.full((16,), total, jnp.int32)

        def spk_blk(b, _):
            wj = b * 16 + lanes
            e0 = jnp.minimum(wj * 2, _L) * 16
            e1 = jnp.minimum(wj * 2 + 1, _L) * 16
            lo = plsc.load_gather(seg, [e0])
            hi = plsc.load_gather(seg, [e1])
            sbuf[pl.ds(b * 16, 16)] = lo | lax.shift_left(hi, 16)
            return 0
        lax.fori_loop(0, _SPW // 16, spk_blk, 0)
        pltpu.sync_copy(sbuf, spk_hbm.at[t])

        def qscat(k, _):
            mv = lanes * (_M // 16) + k
            lv = plsc.load_gather(qlv, [mv])
            a = lv * 16 + lanes
            cur = plsc.load_gather(seg, [a])
            plsc.store_scatter(out1k, [cur], mv)
            plsc.store_scatter(seg, [a], cur + 1)
            return 0
        lax.fori_loop(0, _M // 16, qscat, 0)
        pltpu.sync_copy(out1k, qperm_hbm.at[t])

        def zero_blk2(b, _):
            for u in range(8):
                cnt2[pl.ds((b * 8 + u) * 16, 16)] = z16
            return 0
        lax.fori_loop(0, _L * 2 // 8, zero_blk2, 0)

        def thist_step(k2, _):
            k = k2 * 2
            lv0 = plsc.load_gather(tlv, [lanes * (_N // 16) + k])
            lv1 = plsc.load_gather(tlv, [lanes * (_N // 16) + k + 1])
            a0 = lv0 * 32 + lanes * 2
            a1 = lv1 * 32 + lanes * 2 + 1
            c0 = plsc.load_gather(cnt2, [a0])
            c1 = plsc.load_gather(cnt2, [a1])
            plsc.store_scatter(cnt2, [a0], c0 + 1)
            plsc.store_scatter(cnt2, [a1], c1 + 1)
            return 0
        lax.fori_loop(0, _N // 32, thist_step, 0)

        def tfold(b, _):
            l16 = b * 16 + lanes
            acc = plsc.load_gather(cnt2, [l16 * 32])
            def add_lane(j, acc):
                return acc + plsc.load_gather(cnt2, [l16 * 32 + j])
            acc = lax.fori_loop(1, 32, add_lane, acc)
            out512[pl.ds(b * 16, 16)] = acc
            return 0
        lax.fori_loop(0, _L // 16, tfold, 0)
        pltpu.sync_copy(out512, thist_hbm.at[t])
        return 0

    lax.fori_loop(0, _TREES_PER_TILE, do_tree, 0)


def _rowscale_body(ql_hbm, thist_hbm, rs_hbm, qlb, th, out32):
    w = _wid()
    mchunk = _M // _NTILES
    pltpu.sync_copy(thist_hbm, th)
    pltpu.sync_copy(ql_hbm.at[:, pl.ds(w * mchunk, mchunk)], qlb)

    def per_half(j):
        def per_tree(t, acc):
            lv = qlb[t, pl.ds(j * 16, 16)]
            h = plsc.load_gather(th, [jnp.full((16,), t, jnp.int32), lv])
            return acc + h
        acc = lax.fori_loop(0, _T, per_tree, jnp.zeros((16,), jnp.int32))
        denom = acc.astype(jnp.float32) + jnp.float32(_T * 1e-8)
        out32[pl.ds(j * 16, 16)] = jnp.float32(1.0) / denom

    per_half(0)
    per_half(1)
    pltpu.sync_copy(out32, rs_hbm.at[pl.ds(w * mchunk, mchunk)])


def _qdata_body(qperm_hbm, rs_hbm, qdata_hbm, qpv, rsv, qdv):
    w = _wid()
    pltpu.sync_copy(rs_hbm, rsv)

    def do_tree(i, _):
        t = w * _TREES_PER_TILE + i
        pltpu.sync_copy(qperm_hbm.at[t], qpv)

        def step(j, _):
            mv = qpv[pl.ds(j * 16, 16)]
            rb = plsc.bitcast(plsc.load_gather(rsv, [mv]), jnp.int32)
            rb = (rb + 0x8000) & jnp.int32(-65536)
            qdv[pl.ds(j * 16, 16)] = rb | mv
            return 0
        lax.fori_loop(0, _M // 16, step, 0)
        pltpu.sync_copy(qdv, qdata_hbm.at[t])
        return 0

    lax.fori_loop(0, _TREES_PER_TILE, do_tree, 0)


def _join_body(ql_hbm, tl_hbm, qperm_hbm, spk_hbm, thist_hbm,
               out_hbm, rs_stage, qdata_stage,
               qdata, spk, acc, tlb, qlb, qpv, rsl, rsv, osem, tsem):
    c = lax.axis_index("c")
    sidx = lax.axis_index("s")
    w = c * 16 + sidx
    lanes = _iota16()
    pltpu.sync_copy(spk_hbm, spk)

    pltpu.sync_copy(thist_hbm, qdata.at[:, pl.ds(0, _L)])
    pltpu.sync_copy(ql_hbm.at[:, pl.ds(sidx * 64, 64)], qlb)

    for j in range(4):
        def per_tree_rs(t, accv):
            lv = qlb[t, pl.ds(j * 16, 16)]
            h = plsc.load_gather(qdata, [jnp.full((16,), t, jnp.int32), lv])
            return accv + h
        accv = lax.fori_loop(0, _T, per_tree_rs, jnp.zeros((16,), jnp.int32))
        denom = accv.astype(jnp.float32) + jnp.float32(_T * 1e-8)
        rsl[pl.ds(j * 16, 16)] = jnp.float32(1.0) / denom
    pltpu.sync_copy(rsl, rs_stage.at[c, pl.ds(sidx * 64, 64)])
    plsc.subcore_barrier()

    pltpu.sync_copy(rs_stage.at[c], rsv)
    pltpu.sync_copy(qperm_hbm.at[pl.ds(sidx * 4, 4)], qpv)
    for i in range(4):
        def pack_step(j, _):
            mv = qpv[i, pl.ds(j * 16, 16)]
            rb = plsc.bitcast(plsc.load_gather(rsv, [mv]), jnp.int32)
            rb = (rb + 0x8000) & jnp.int32(-65536)
            qpv[i, pl.ds(j * 16, 16)] = rb | mv
            return 0
        lax.fori_loop(0, _M // 16, pack_step, 0)
    pltpu.sync_copy(qpv, qdata_stage.at[c, pl.ds(sidx * 4, 4)])
    plsc.subcore_barrier()

    pltpu.sync_copy(qdata_stage.at[c], qdata)

    npass = (_NSTRIPS + _NTILES - 1) // _NTILES

    def strip_of(si):
        return w + si * _NTILES

    def tl_copy(si):
        return pltpu.make_async_copy(
            tl_hbm.at[:, pl.ds(strip_of(si) * _W, _W)], tlb, tsem)

    def out_copy(si, b):
        return pltpu.make_async_copy(
            acc.at[b], out_hbm.at[:, pl.ds(strip_of(si) * _W, _W)],
            osem.at[b])


    def do_pass(p, _):
        for h in range(2):
            si = p * 2 + h
            b = h

            @pl.when(strip_of(si) < _NSTRIPS)
            def _():
                acc_b = acc.at[b]

                tl_copy(si).start()

                @pl.when(si >= 2)
                def _():
                    out_copy(si - 2, b).wait()

                def zero_rows(r, _):
                    for rr in range(16):
                        acc_b[r * 16 + rr, :] = jnp.zeros((16,), jnp.float32)
                    return 0
                lax.fori_loop(0, _M // 16, zero_rows, 0)
                tl_copy(si).wait()

                def header(t):
                    tlv = tlb[t, :]
                    trow = jnp.full((16,), t, jnp.int32)
                    wv = lax.shift_right_logical(tlv, 1)
                    odd = (tlv & 1) > 0
                    u0 = plsc.load_gather(spk, [trow, wv])
                    u1 = plsc.load_gather(spk, [trow, wv + 1])
                    lo0 = u0 & 0xFFFF
                    hi0 = lax.shift_right_logical(u0, 16)
                    lo1 = u1 & 0xFFFF
                    start = jnp.where(odd, hi0, lo0)
                    ln = jnp.where(odd, lo1, hi0) - start
                    return trow, start, ln

                def per_quad(tq, _):
                    hs = [header(tq * 4 + i) for i in range(4)]
                    mx = jnp.max(jnp.maximum(
                        jnp.maximum(hs[0][2], hs[1][2]),
                        jnp.maximum(hs[2][2], hs[3][2])))

                    def expand(k2, _):
                        k = k2 * 2
                        qs = []
                        for trow, start, ln in hs:
                            qs.append((plsc.load_gather(
                                qdata, [trow, start + k]), ln > k))
                            qs.append((plsc.load_gather(
                                qdata, [trow, start + k + 1]), ln > k + 1))
                        for q, msk in qs:
                            plsc.addupdate_scatter(
                                acc_b, [q & 0xFFFF, lanes],
                                plsc.bitcast(q & jnp.int32(-65536),
                                             jnp.float32),
                                mask=msk)
                        return 0
                    lax.fori_loop(0, (mx + 1) // 2, expand, 0)
                    return 0
                lax.fori_loop(0, _T // 4, per_quad, 0)
                out_copy(si, b).start()
        return 0

    lax.fori_loop(0, (npass + 1) // 2, do_pass, 0)

    nstrips = _NSTRIPS // _NTILES + jnp.where(w < _NSTRIPS % _NTILES, 1, 0)

    @pl.when(nstrips >= 2)
    def _():
        out_copy(nstrips - 2, (nstrips - 2) % 2).wait()

    @pl.when(nstrips >= 1)
    def _():
        out_copy(nstrips - 1, (nstrips - 1) % 2).wait()


def _sc_pipeline(ql, tl):
    mesh = plsc.VectorSubcoreMesh(core_axis_name="c", subcore_axis_name="s")

    csr = pl.kernel(
        _csr_body,
        mesh=mesh,
        compiler_params=pltpu.CompilerParams(needs_layout_passes=False, use_tc_tiling_on_sc=False),
        out_type=(
            jax.ShapeDtypeStruct((_T, _M), jnp.int32),
            jax.ShapeDtypeStruct((_T, _SPW), jnp.int32),
            jax.ShapeDtypeStruct((_T, _L), jnp.int32),
        ),
        scratch_types=[
            pltpu.VMEM((_M,), jnp.int32),
            pltpu.VMEM((_N,), jnp.int32),
            pltpu.VMEM((_L * 16,), jnp.int32),
            pltpu.VMEM((_L * 32,), jnp.int32),
            pltpu.VMEM((_L * 16 + 16,), jnp.int32),
            pltpu.VMEM((_M,), jnp.int32),
            pltpu.VMEM((_L,), jnp.int32),
            pltpu.VMEM((_SPW,), jnp.int32),
        ],
    )
    qperm, spk, thist = csr(ql, tl)

    out, _, _ = pl.kernel(
        _join_body,
        mesh=mesh,
        compiler_params=pltpu.CompilerParams(needs_layout_passes=False, use_tc_tiling_on_sc=False),
        out_type=(
            jax.ShapeDtypeStruct((_M, _N), jnp.float32),
            jax.ShapeDtypeStruct((2, _M), jnp.float32),
            jax.ShapeDtypeStruct((2, _T, _M), jnp.int32),
        ),
        scratch_types=[
            pltpu.VMEM((_T, _M), jnp.int32),
            pltpu.VMEM((_T, _SPW), jnp.int32),
            pltpu.VMEM((2, _M, _W), jnp.float32),
            pltpu.VMEM((_T, _W), jnp.int32),
            pltpu.VMEM((_T, 64), jnp.int32),
            pltpu.VMEM((4, _M), jnp.int32),
            pltpu.VMEM((64,), jnp.float32),
            pltpu.VMEM((_M,), jnp.float32),
            pltpu.SemaphoreType.DMA((2,)),
            pltpu.SemaphoreType.DMA,
        ],
    )(ql, tl, qperm, spk, thist)
    return out


def kernel(X, query_leaves, train_leaves):
    del X
    return _sc_pipeline(query_leaves, train_leaves)


# speedup vs baseline: 10.5604x; 1.0243x over previous
import functools

import jax
import jax.numpy as jnp
from jax import lax
from jax.experimental import pallas as pl
from jax.experimental.pallas import tpu as pltpu
from jax.experimental.pallas import tpu_sc as plsc

_T = 64
_M = 1024
_N = 20000
_L = 512

_NTILES = 32
_TREES_PER_TILE = _T // _NTILES
_W = 16
_NSTRIPS = _N // _W
_SPW = 272
_TG = 8


def _iota16():
    return lax.iota(jnp.int32, 16)


def _wid():
    return lax.axis_index("c") * 16 + lax.axis_index("s")


def _csr_body(ql_hbm, tl_hbm, qperm_hbm, spk_hbm, thist_hbm,
              qlv, tlv, cnt, cnt2, seg, out1k, out512, sbuf):
    w = _wid()
    lanes = _iota16()

    def do_tree(i, _):
        t = w * _TREES_PER_TILE + i
        pltpu.sync_copy(ql_hbm.at[t], qlv)
        pltpu.sync_copy(tl_hbm.at[t], tlv)

        z16 = jnp.zeros((16,), jnp.int32)

        def zero_blk(b, _):
            for u in range(8):
                cnt[pl.ds((b * 8 + u) * 16, 16)] = z16
            return 0
        lax.fori_loop(0, _L // 8, zero_blk, 0)

        def qhist(k, _):
            lv = plsc.load_gather(qlv, [lanes * (_M // 16) + k])
            a = lv * 16 + lanes
            c = plsc.load_gather(cnt, [a])
            plsc.store_scatter(cnt, [a], c + 1)
            return 0
        lax.fori_loop(0, _M // 16, qhist, 0)

        def scan_blk(b2, carry):
            b = b2 * 2
            v0 = cnt[pl.ds(b * 16, 16)]
            v1 = cnt[pl.ds(b * 16 + 16, 16)]
            inc0 = plsc.cumsum(v0)
            inc1 = plsc.cumsum(v1)
            carry0 = carry + inc0[15]
            seg[pl.ds(b * 16, 16)] = inc0 - v0 + carry
            seg[pl.ds(b * 16 + 16, 16)] = inc1 - v1 + carry0
            return carry0 + inc1[15]
        total = lax.fori_loop(0, _L // 2, scan_blk, jnp.int32(0))
        seg[pl.ds(_L * 16, 16)] = jnp.full((16,), total, jnp.int32)

        def spk_blk(b, _):
            wj = b * 16 + lanes
            e0 = jnp.minimum(wj * 2, _L) * 16
            e1 = jnp.minimum(wj * 2 + 1, _L) * 16
            lo = plsc.load_gather(seg, [e0])
            hi = plsc.load_gather(seg, [e1])
            sbuf[pl.ds(b * 16, 16)] = lo | lax.shift_left(hi, 16)
            return 0
        lax.fori_loop(0, _SPW // 16, spk_blk, 0)
        pltpu.sync_copy(sbuf, spk_hbm.at[t])

        def qscat(k, _):
            mv = lanes * (_M // 16) + k
            lv = plsc.load_gather(qlv, [mv])
            a = lv * 16 + lanes
            cur = plsc.load_gather(seg, [a])
            plsc.store_scatter(out1k, [cur], mv)
            plsc.store_scatter(seg, [a], cur + 1)
            return 0
        lax.fori_loop(0, _M // 16, qscat, 0)
        pltpu.sync_copy(out1k, qperm_hbm.at[t])

        def zero_blk2(b, _):
            for u in range(8):
                cnt2[pl.ds((b * 8 + u) * 16, 16)] = z16
            return 0
        lax.fori_loop(0, _L * 2 // 8, zero_blk2, 0)

        def thist_step(k2, _):
            k = k2 * 2
            lv0 = plsc.load_gather(tlv, [lanes * (_N // 16) + k])
            lv1 = plsc.load_gather(tlv, [lanes * (_N // 16) + k + 1])
            a0 = lv0 * 32 + lanes * 2
            a1 = lv1 * 32 + lanes * 2 + 1
            c0 = plsc.load_gather(cnt2, [a0])
            c1 = plsc.load_gather(cnt2, [a1])
            plsc.store_scatter(cnt2, [a0], c0 + 1)
            plsc.store_scatter(cnt2, [a1], c1 + 1)
            return 0
        lax.fori_loop(0, _N // 32, thist_step, 0)

        def tfold(b, _):
            l16 = b * 16 + lanes
            acc = plsc.load_gather(cnt2, [l16 * 32])
            def add_lane(j, acc):
                return acc + plsc.load_gather(cnt2, [l16 * 32 + j])
            acc = lax.fori_loop(1, 32, add_lane, acc)
            out512[pl.ds(b * 16, 16)] = acc
            return 0
        lax.fori_loop(0, _L // 16, tfold, 0)
        pltpu.sync_copy(out512, thist_hbm.at[t])
        return 0

    lax.fori_loop(0, _TREES_PER_TILE, do_tree, 0)


def _rowscale_body(ql_hbm, thist_hbm, rs_hbm, qlb, th, out32):
    w = _wid()
    mchunk = _M // _NTILES
    pltpu.sync_copy(thist_hbm, th)
    pltpu.sync_copy(ql_hbm.at[:, pl.ds(w * mchunk, mchunk)], qlb)

    def per_half(j):
        def per_tree(t, acc):
            lv = qlb[t, pl.ds(j * 16, 16)]
            h = plsc.load_gather(th, [jnp.full((16,), t, jnp.int32), lv])
            return acc + h
        acc = lax.fori_loop(0, _T, per_tree, jnp.zeros((16,), jnp.int32))
        denom = acc.astype(jnp.float32) + jnp.float32(_T * 1e-8)
        out32[pl.ds(j * 16, 16)] = jnp.float32(1.0) / denom

    per_half(0)
    per_half(1)
    pltpu.sync_copy(out32, rs_hbm.at[pl.ds(w * mchunk, mchunk)])


def _qdata_body(qperm_hbm, rs_hbm, qdata_hbm, qpv, rsv, qdv):
    w = _wid()
    pltpu.sync_copy(rs_hbm, rsv)

    def do_tree(i, _):
        t = w * _TREES_PER_TILE + i
        pltpu.sync_copy(qperm_hbm.at[t], qpv)

        def step(j, _):
            mv = qpv[pl.ds(j * 16, 16)]
            rb = plsc.bitcast(plsc.load_gather(rsv, [mv]), jnp.int32)
            rb = (rb + 0x8000) & jnp.int32(-65536)
            qdv[pl.ds(j * 16, 16)] = rb | mv
            return 0
        lax.fori_loop(0, _M // 16, step, 0)
        pltpu.sync_copy(qdv, qdata_hbm.at[t])
        return 0

    lax.fori_loop(0, _TREES_PER_TILE, do_tree, 0)


def _join_body(ql_hbm, tl_hbm, qperm_hbm, spk_hbm, thist_hbm,
               out_hbm, rs_stage, qdata_stage,
               qdata, spk, acc, tlb, qlb, qpv, rsl, rsv, osem, tsem):
    c = lax.axis_index("c")
    sidx = lax.axis_index("s")
    w = c * 16 + sidx
    lanes = _iota16()
    pltpu.sync_copy(spk_hbm, spk)

    pltpu.sync_copy(thist_hbm, qdata.at[:, pl.ds(0, _L)])
    pltpu.sync_copy(ql_hbm.at[:, pl.ds(sidx * 64, 64)], qlb)

    for j in range(4):
        def per_tree_rs(t, accv):
            lv = qlb[t, pl.ds(j * 16, 16)]
            h = plsc.load_gather(qdata, [jnp.full((16,), t, jnp.int32), lv])
            return accv + h
        accv = lax.fori_loop(0, _T, per_tree_rs, jnp.zeros((16,), jnp.int32))
        denom = accv.astype(jnp.float32) + jnp.float32(_T * 1e-8)
        rsl[pl.ds(j * 16, 16)] = jnp.float32(1.0) / denom
    pltpu.sync_copy(rsl, rs_stage.at[c, pl.ds(sidx * 64, 64)])
    plsc.subcore_barrier()

    pltpu.sync_copy(rs_stage.at[c], rsv)
    pltpu.sync_copy(qperm_hbm.at[pl.ds(sidx * 4, 4)], qpv)
    for i in range(4):
        def pack_step(j, _):
            mv = qpv[i, pl.ds(j * 16, 16)]
            rb = plsc.bitcast(plsc.load_gather(rsv, [mv]), jnp.int32)
            rb = (rb + 0x8000) & jnp.int32(-65536)
            qpv[i, pl.ds(j * 16, 16)] = rb | mv
            return 0
        lax.fori_loop(0, _M // 16, pack_step, 0)
    pltpu.sync_copy(qpv, qdata_stage.at[c, pl.ds(sidx * 4, 4)])
    plsc.subcore_barrier()

    pltpu.sync_copy(qdata_stage.at[c], qdata)

    npass = (_NSTRIPS + _NTILES - 1) // _NTILES

    def strip_of(si):
        return w + si * _NTILES

    def tl_copy(si):
        return pltpu.make_async_copy(
            tl_hbm.at[:, pl.ds(strip_of(si) * _W, _W)], tlb, tsem)

    def out_copy(si, b):
        return pltpu.make_async_copy(
            acc.at[b], out_hbm.at[:, pl.ds(strip_of(si) * _W, _W)],
            osem.at[b])


    def do_pass(p, _):
        for h in range(2):
            si = p * 2 + h
            b = h

            @pl.when(strip_of(si) < _NSTRIPS)
            def _():
                acc_b = acc.at[b]

                tl_copy(si).start()

                @pl.when(si >= 2)
                def _():
                    out_copy(si - 2, b).wait()

                def zero_rows(r, _):
                    for rr in range(16):
                        acc_b[r * 16 + rr, :] = jnp.zeros((16,), jnp.float32)
                    return 0
                lax.fori_loop(0, _M // 16, zero_rows, 0)
                tl_copy(si).wait()

                def header(t):
                    tlv = tlb[t, :]
                    trow = jnp.full((16,), t, jnp.int32)
                    wv = lax.shift_right_logical(tlv, 1)
                    odd = (tlv & 1) > 0
                    u0 = plsc.load_gather(spk, [trow, wv])
                    u1 = plsc.load_gather(spk, [trow, wv + 1])
                    lo0 = u0 & 0xFFFF
                    hi0 = lax.shift_right_logical(u0, 16)
                    lo1 = u1 & 0xFFFF
                    start = jnp.where(odd, hi0, lo0)
                    ln = jnp.where(odd, lo1, hi0) - start
                    return trow, start, ln

                def per_quad(tq, _):
                    hs = [header(tq * _TG + i) for i in range(_TG)]
                    lnmax = hs[0][2]
                    for _, _, lni in hs[1:]:
                        lnmax = jnp.maximum(lnmax, lni)
                    mx = jnp.max(lnmax)

                    def expand(k2, _):
                        k = k2 * 2
                        qs = []
                        for trow, start, ln in hs:
                            qs.append((plsc.load_gather(
                                qdata, [trow, start + k]), ln > k))
                            qs.append((plsc.load_gather(
                                qdata, [trow, start + k + 1]), ln > k + 1))
                        for q, msk in qs:
                            plsc.addupdate_scatter(
                                acc_b, [q & 0xFFFF, lanes],
                                plsc.bitcast(q & jnp.int32(-65536),
                                             jnp.float32),
                                mask=msk)
                        return 0
                    lax.fori_loop(0, (mx + 1) // 2, expand, 0)
                    return 0
                lax.fori_loop(0, _T // _TG, per_quad, 0)
                out_copy(si, b).start()
        return 0

    lax.fori_loop(0, (npass + 1) // 2, do_pass, 0)

    nstrips = _NSTRIPS // _NTILES + jnp.where(w < _NSTRIPS % _NTILES, 1, 0)

    @pl.when(nstrips >= 2)
    def _():
        out_copy(nstrips - 2, (nstrips - 2) % 2).wait()

    @pl.when(nstrips >= 1)
    def _():
        out_copy(nstrips - 1, (nstrips - 1) % 2).wait()


def _sc_pipeline(ql, tl):
    mesh = plsc.VectorSubcoreMesh(core_axis_name="c", subcore_axis_name="s")

    csr = pl.kernel(
        _csr_body,
        mesh=mesh,
        compiler_params=pltpu.CompilerParams(needs_layout_passes=False, use_tc_tiling_on_sc=False),
        out_type=(
            jax.ShapeDtypeStruct((_T, _M), jnp.int32),
            jax.ShapeDtypeStruct((_T, _SPW), jnp.int32),
            jax.ShapeDtypeStruct((_T, _L), jnp.int32),
        ),
        scratch_types=[
            pltpu.VMEM((_M,), jnp.int32),
            pltpu.VMEM((_N,), jnp.int32),
            pltpu.VMEM((_L * 16,), jnp.int32),
            pltpu.VMEM((_L * 32,), jnp.int32),
            pltpu.VMEM((_L * 16 + 16,), jnp.int32),
            pltpu.VMEM((_M,), jnp.int32),
            pltpu.VMEM((_L,), jnp.int32),
            pltpu.VMEM((_SPW,), jnp.int32),
        ],
    )
    qperm, spk, thist = csr(ql, tl)

    out, _, _ = pl.kernel(
        _join_body,
        mesh=mesh,
        compiler_params=pltpu.CompilerParams(needs_layout_passes=False, use_tc_tiling_on_sc=False),
        out_type=(
            jax.ShapeDtypeStruct((_M, _N), jnp.float32),
            jax.ShapeDtypeStruct((2, _M), jnp.float32),
            jax.ShapeDtypeStruct((2, _T, _M), jnp.int32),
        ),
        scratch_types=[
            pltpu.VMEM((_T, _M), jnp.int32),
            pltpu.VMEM((_T, _SPW), jnp.int32),
            pltpu.VMEM((2, _M, _W), jnp.float32),
            pltpu.VMEM((_T, _W), jnp.int32),
            pltpu.VMEM((_T, 64), jnp.int32),
            pltpu.VMEM((4, _M), jnp.int32),
            pltpu.VMEM((64,), jnp.float32),
            pltpu.VMEM((_M,), jnp.float32),
            pltpu.SemaphoreType.DMA((2,)),
            pltpu.SemaphoreType.DMA,
        ],
    )(ql, tl, qperm, spk, thist)
    return out


def kernel(X, query_leaves, train_leaves):
    del X
    return _sc_pipeline(query_leaves, train_leaves)
